# baseline (speedup 1.0000x reference)
import functools
import numpy as np
import jax
import jax.numpy as jnp
from jax import lax
from jax.experimental import pallas as pl
from jax.experimental.pallas import tpu as pltpu
from jax.experimental.pallas import tpu_sc as plsc

F = 16
S = 10
EPS = 0.25
_SQ3 = float(np.sqrt(3.0))

_NC = 2
_NS = 16
_IDXW = 128
_CHUNK_ROWS = 8
_CE = _IDXW
_NPAD = 50048


def _up_body(nf0_ref, nf1_ref, w0_ref, w1_ref, out_ref):
    w0 = w0_ref[...]
    w1 = w1_ref[...]
    h0 = jnp.dot(nf0_ref[...], w0, preferred_element_type=jnp.float32)
    nf1 = nf1_ref[...]
    h1x = jnp.dot(nf1[:, 0:16], w1, preferred_element_type=jnp.float32)
    h1y = jnp.dot(nf1[:, 16:32], w1, preferred_element_type=jnp.float32)
    h1z = jnp.dot(nf1[:, 32:48], w1, preferred_element_type=jnp.float32)
    out_ref[...] = jnp.concatenate([h0, h1x, h1y, h1z], axis=1)


def _up_call(nf0, nf1p, W_up_l0, W_up_l1):
    n = nf0.shape[0]
    nb = 2000
    grid = n // nb
    return pl.pallas_call(
        _up_body,
        grid=(grid,),
        in_specs=[
            pl.BlockSpec((nb, F), lambda i: (i, 0)),
            pl.BlockSpec((nb, 3 * F), lambda i: (i, 0)),
            pl.BlockSpec((F, F), lambda i: (0, 0)),
            pl.BlockSpec((F, F), lambda i: (0, 0)),
        ],
        out_specs=pl.BlockSpec((nb, 4 * F), lambda i: (i, 0)),
        out_shape=jax.ShapeDtypeStruct((n, 4 * F), jnp.float32),
    )(nf0, nf1p, W_up_l0, W_up_l1)


def _silu(x):
    return x * jax.nn.sigmoid(x)


def _edge_body(nvalid, r_ref, m0_ref, m1_ref, m2_ref, m3_ref, out_ref):
    r = r_ref[...]
    h = _silu(jnp.dot(r, m0_ref[...], preferred_element_type=jnp.float32))
    h = _silu(jnp.dot(h, m1_ref[...], preferred_element_type=jnp.float32))
    h = _silu(jnp.dot(h, m2_ref[...], preferred_element_type=jnp.float32))
    mix = jnp.dot(h, m3_ref[...], preferred_element_type=jnp.float32)
    wscale = jnp.concatenate([
        jnp.ones((16,), jnp.float32),
        jnp.full((16,), 1.0 / _SQ3, jnp.float32),
        jnp.ones((48,), jnp.float32),
    ])[None, :]
    out_ref[...] = mix * wscale
    eb = r_ref.shape[0]
    base = pl.program_id(0) * eb

    @pl.when(base + eb > nvalid)
    def _():
        rid = base + jax.lax.broadcasted_iota(jnp.int32, (eb, 80), 0)
        out_ref[...] = jnp.where(rid < nvalid, out_ref[...], 0.0)


def _edge_call(r8, epad, mlp_w0, mlp_w1, mlp_w2, mlp_w3):
    e = r8.shape[0]
    eb = 2048
    grid = epad // eb
    last_in = (e - 1) // eb
    return pl.pallas_call(
        functools.partial(_edge_body, e),
        grid=(grid,),
        in_specs=[
            pl.BlockSpec((eb, 8), lambda i: (jnp.minimum(i, last_in), 0)),
            pl.BlockSpec(mlp_w0.shape, lambda i: (0, 0)),
            pl.BlockSpec(mlp_w1.shape, lambda i: (0, 0)),
            pl.BlockSpec(mlp_w2.shape, lambda i: (0, 0)),
            pl.BlockSpec(mlp_w3.shape, lambda i: (0, 0)),
        ],
        out_specs=pl.BlockSpec((eb, 80), lambda i: (i, 0)),
        out_shape=jax.ShapeDtypeStruct((epad, 80), jnp.float32),
    )(r8, mlp_w0, mlp_w1, mlp_w2, mlp_w3)


def _sh_body(nvalid, v_ref, out_ref):
    vT = v_ref[...].T
    x = vT[0:1, :]
    y = vT[1:2, :]
    z = vT[2:3, :]
    rn = jnp.sqrt(x * x + y * y + z * z)
    inv = 1.0 / (rn + 1e-9)
    ux = x * inv
    uy = y * inv
    uz = z * inv
    shx = _SQ3 * ux
    shy = _SQ3 * uy
    shz = _SQ3 * uz
    a = _SQ3 * ux * uy
    b = _SQ3 * uy * uz
    c = 1.5 * uz * uz - 0.5
    d = _SQ3 * ux * uz
    e = 0.5 * _SQ3 * (ux * ux - uy * uy)
    m00 = e - 0.5 * c
    m11 = -e - 0.5 * c
    zpad = jnp.zeros((7, x.shape[1]), jnp.float32)
    scalT = jnp.concatenate(
        [shx, shy, shz, m00, a, d, m11, b, c, zpad], axis=0)
    out_ref[...] = scalT.T
    ebT = v_ref.shape[0]
    base = pl.program_id(0) * ebT

    @pl.when(base + ebT > nvalid)
    def _():
        rid = base + jax.lax.broadcasted_iota(jnp.int32, (ebT, 16), 0)
        out_ref[...] = jnp.where(rid < nvalid, out_ref[...], 0.0)


def _sh_call(vec, epad):
    e = vec.shape[0]
    ebT = 16384
    grid = epad // ebT
    return pl.pallas_call(
        functools.partial(_sh_body, e),
        grid=(grid,),
        in_specs=[pl.BlockSpec((ebT, 3), lambda i: (i, 0))],
        out_specs=pl.BlockSpec((ebT, 16), lambda i: (i, 0)),
        out_shape=jax.ShapeDtypeStruct((epad, 16), jnp.float32),
    )(vec)


def _sc_body(h_hbm, p_hbm, s_hbm, snd_hbm, rcv_hbm, ya_hbm, yb_hbm,
             acc_sh, idx_s, idx_r, p_v, s_v, x_v, m_v, z_v,
             sem_p, sem_s, sem_g):
    core = lax.axis_index("c")
    sub = lax.axis_index("s")
    n = ya_hbm.shape[0]
    rows_per_sub_out = n // _NS
    zrows = z_v.shape[0]

    zero16 = jnp.zeros((16,), jnp.float32)

    def zinit(i, carry):
        z_v[i, 0:16] = zero16
        z_v[i, 16:32] = zero16
        return carry

    lax.fori_loop(0, zrows, zinit, 0)
    obase = sub * rows_per_sub_out

    def zcopy(i, carry):
        pltpu.sync_copy(z_v, acc_sh.at[pl.ds(obase + i * zrows, zrows)])
        return carry

    lax.fori_loop(0, rows_per_sub_out // zrows, zcopy, 0)
    plsc.subcore_barrier()

    total_rows = snd_hbm.shape[0]
    rows_per_sub = total_rows // _NS
    row0 = sub * rows_per_sub

    def compute_core0(e4, carry):
        for j in range(4):
            e = e4 * 4 + j
            x0 = x_v[e, 0:16]
            xx = x_v[e, 16:32]
            xy = x_v[e, 32:48]
            xz = x_v[e, 48:64]
            w0 = p_v[e, 0:16]
            w1 = p_v[e, 16:32]
            w2 = p_v[e, 32:48]
            w3 = p_v[e, 48:64]
            w4 = p_v[e, 64:80]
            sv = s_v[e, 0:16]
            shx = sv[0]
            shy = sv[1]
            shz = sv[2]
            m00 = sv[3]
            m01 = sv[4]
            m02 = sv[5]
            dot = xx * shx + xy * shy + xz * shz
            m0 = w0 * x0 + w1 * dot
            t = w2 * x0
            m1x = t * shx + w3 * xx + w4 * (xx * m00 + xy * m01 + xz * m02)
            m_v[e, 0:16] = m0
            m_v[e, 16:32] = m1x
        return carry

    def compute_core1(e4, carry):
        for j in range(4):
            e = e4 * 4 + j
            x0 = x_v[e, 0:16]
            xx = x_v[e, 16:32]
            xy = x_v[e, 32:48]
            xz = x_v[e, 48:64]
            w2 = p_v[e, 32:48]
            w3 = p_v[e, 48:64]
            w4 = p_v[e, 64:80]
            sv = s_v[e, 0:16]
            shy = sv[1]
            shz = sv[2]
            m01 = sv[4]
            m02 = sv[5]
            m11 = sv[6]
            m12 = sv[7]
            m22 = sv[8]
            t = w2 * x0
            m1y = t * shy + w3 * xy + w4 * (xx * m01 + xy * m11 + xz * m12)
            m1z = t * shz + w3 * xz + w4 * (xx * m02 + xy * m12 + xz * m22)
            m_v[e, 0:16] = m1y
            m_v[e, 16:32] = m1z
        return carry

    def chunk(k, carry):
        rbase = row0 + k * _CHUNK_ROWS
        pltpu.sync_copy(snd_hbm.at[pl.ds(rbase, _CHUNK_ROWS)], idx_s)
        pltpu.sync_copy(rcv_hbm.at[pl.ds(rbase, _CHUNK_ROWS)], idx_r)
        for h in range(_CHUNK_ROWS):
            cp = pltpu.async_copy(
                p_hbm.at[pl.ds((rbase + h) * _IDXW, _CE)], p_v, sem_p)
            cs = pltpu.async_copy(
                s_hbm.at[pl.ds((rbase + h) * _IDXW, _CE)], s_v, sem_s)
            gd = pltpu.async_copy(h_hbm.at[idx_s.at[h]], x_v, sem_g)
            cp.wait()
            cs.wait()
            gd.wait()

            @pl.when(core == 0)
            def _():
                lax.fori_loop(0, _CE // 4, compute_core0, 0)

            @pl.when(core == 1)
            def _():
                lax.fori_loop(0, _CE // 4, compute_core1, 0)

            pltpu.sync_copy(m_v, acc_sh.at[idx_r.at[h]], add=True)
        return carry

    lax.fori_loop(0, rows_per_sub // _CHUNK_ROWS, chunk, 0)
    plsc.subcore_barrier()

    @pl.when(core == 0)
    def _():
        pltpu.sync_copy(acc_sh.at[pl.ds(obase, rows_per_sub_out)],
                        ya_hbm.at[pl.ds(obase, rows_per_sub_out)])

    @pl.when(core == 1)
    def _():
        pltpu.sync_copy(acc_sh.at[pl.ds(obase, rows_per_sub_out)],
                        yb_hbm.at[pl.ds(obase, rows_per_sub_out)])


def _sc_call(h_tab, p_edge, s_edge, snd2, rcv2):
    mesh = plsc.VectorSubcoreMesh(
        core_axis_name="c", subcore_axis_name="s",
        num_cores=_NC, num_subcores=_NS)
    fn = functools.partial(
        pl.kernel,
        out_type=[
            jax.ShapeDtypeStruct((_NPAD, 32), jnp.float32),
            jax.ShapeDtypeStruct((_NPAD, 32), jnp.float32),
        ],
        mesh=mesh,
        scratch_types=[
            pltpu.VMEM_SHARED((_NPAD, 32), jnp.float32),
            pltpu.VMEM((_CHUNK_ROWS, _IDXW), jnp.int32),
            pltpu.VMEM((_CHUNK_ROWS, _IDXW), jnp.int32),
            pltpu.VMEM((_CE, 80), jnp.float32),
            pltpu.VMEM((_CE, 16), jnp.float32),
            pltpu.VMEM((_CE, 64), jnp.float32),
            pltpu.VMEM((_CE, 32), jnp.float32),
            pltpu.VMEM((8, 32), jnp.float32),
            pltpu.SemaphoreType.DMA,
            pltpu.SemaphoreType.DMA,
            pltpu.SemaphoreType.DMA,
        ],
        compiler_params=pltpu.CompilerParams(use_tc_tiling_on_sc=False),
    )(_sc_body)
    return fn(h_tab, p_edge, s_edge, snd2, rcv2)


def _node_body(ya_ref, yb_ref, nf0_ref, nf1_ref,
               wd0_ref, wd1_ref, wsk0_ref, wsk1_ref, wsc_ref,
               wp0_ref, wp1_ref, wro_ref,
               z0_ref, z1_ref, ro_ref):
    ya = ya_ref[...]
    yb = yb_ref[...]
    wd0 = wd0_ref[...]
    wd1 = wd1_ref[...]
    s = jnp.dot(ya[:, 0:16], wd0, preferred_element_type=jnp.float32)
    v1x = jnp.dot(ya[:, 16:32], wd1, preferred_element_type=jnp.float32)
    v1y = jnp.dot(yb[:, 0:16], wd1, preferred_element_type=jnp.float32)
    v1z = jnp.dot(yb[:, 16:32], wd1, preferred_element_type=jnp.float32)
    n2 = v1x * v1x + v1y * v1y + v1z * v1z
    wz = wsc_ref[0]
    s2 = s * s
    z0 = (wz[0:1, :] * s + wz[1:2, :] * s2 + wz[2:3, :] * (s2 * s)
          + wz[3:4, :] * n2 + wz[4:5, :] * (s * n2))
    t1 = wz[5:6, :] + wz[6:7, :] * s + wz[7:8, :] * s2 + wz[8:9, :] * n2
    z1x = t1 * v1x
    z1y = t1 * v1y
    z1z = t1 * v1z
    wp0 = wp0_ref[...]
    wp1 = wp1_ref[...]
    z0 = jnp.dot(z0, wp0, preferred_element_type=jnp.float32)
    z1x = jnp.dot(z1x, wp1, preferred_element_type=jnp.float32)
    z1y = jnp.dot(z1y, wp1, preferred_element_type=jnp.float32)
    z1z = jnp.dot(z1z, wp1, preferred_element_type=jnp.float32)
    nf1 = nf1_ref[...]
    wsk0 = wsk0_ref[0]
    wsk1 = wsk1_ref[0]
    z0 = z0 + jnp.dot(nf0_ref[...], wsk0, preferred_element_type=jnp.float32)
    z1x = z1x + jnp.dot(nf1[:, 0:16], wsk1, preferred_element_type=jnp.float32)
    z1y = z1y + jnp.dot(nf1[:, 16:32], wsk1,
                        preferred_element_type=jnp.float32)
    z1z = z1z + jnp.dot(nf1[:, 32:48], wsk1,
                        preferred_element_type=jnp.float32)
    z0_ref[...] = z0
    z1_ref[...] = jnp.concatenate([z1x, z1y, z1z], axis=1)
    ro_ref[...] = jnp.sum(z0 * wro_ref[...][:, 0][None, :], axis=1,
                          keepdims=True)


def _node_call(ya, yb, nf0, nf1p, wd0e, wd1e, wsk0, wsk1, wscp, wp0, wp1, wro):
    n = nf0.shape[0]
    nb = min(1000, n // S)
    bps = (n // S) // nb
    grid = n // nb

    return pl.pallas_call(
        _node_body,
        grid=(grid,),
        in_specs=[
            pl.BlockSpec((nb, 32), lambda i: (i, 0)),
            pl.BlockSpec((nb, 32), lambda i: (i, 0)),
            pl.BlockSpec((nb, F), lambda i: (i, 0)),
            pl.BlockSpec((nb, 3 * F), lambda i: (i, 0)),
            pl.BlockSpec((F, F), lambda i: (0, 0)),
            pl.BlockSpec((F, F), lambda i: (0, 0)),
            pl.BlockSpec((1, F, F), lambda i: (i // bps, 0, 0)),
            pl.BlockSpec((1, F, F), lambda i: (i // bps, 0, 0)),
            pl.BlockSpec((1, F, F), lambda i: (i // bps, 0, 0)),
            pl.BlockSpec((F, F), lambda i: (0, 0)),
            pl.BlockSpec((F, F), lambda i: (0, 0)),
            pl.BlockSpec((F, 1), lambda i: (0, 0)),
        ],
        out_specs=[
            pl.BlockSpec((nb, F), lambda i: (i, 0)),
            pl.BlockSpec((nb, 3 * F), lambda i: (i, 0)),
            pl.BlockSpec((nb, 1), lambda i: (i, 0)),
        ],
        out_shape=[
            jax.ShapeDtypeStruct((n, F), jnp.float32),
            jax.ShapeDtypeStruct((n, 3 * F), jnp.float32),
            jax.ShapeDtypeStruct((n, 1), jnp.float32),
        ],
    )(ya, yb, nf0, nf1p, wd0e, wd1e, wsk0, wsk1, wscp, wp0, wp1, wro)


def kernel(vectors, node_feats_l0, node_feats_l1, num_species_counts,
           radial_embeddings, senders, receivers, num_nodes,
           W_up_l0, W_up_l1, mlp_w0, mlp_w1, mlp_w2, mlp_w3,
           W_down_l0, W_down_l1, W_skip_l0, W_skip_l1, W_sc,
           W_post_l0, W_post_l1, W_ro):
    n = node_feats_l0.shape[0]
    e = vectors.shape[0]
    epb = _IDXW * _NS * _CHUNK_ROWS
    e_pad = ((e + epb - 1) // epb) * epb

    nf0 = node_feats_l0[:, :, 0]
    nf1p = jnp.transpose(node_feats_l1, (0, 2, 1)).reshape(n, 3 * F)

    h_tab = _up_call(nf0, nf1p, W_up_l0, W_up_l1)

    p_edge = _edge_call(radial_embeddings, e_pad,
                        mlp_w0, mlp_w1, mlp_w2, mlp_w3)
    s_edge = _sh_call(vectors, e_pad)

    zpad_i = jnp.zeros((e_pad - e,), jnp.int32)
    snd2 = jnp.concatenate([senders.astype(jnp.int32), zpad_i]).reshape(
        e_pad // _IDXW, _IDXW)
    rcv2 = jnp.concatenate([receivers.astype(jnp.int32), zpad_i]).reshape(
        e_pad // _IDXW, _IDXW)

    ya, yb = _sc_call(h_tab, p_edge, s_edge, snd2, rcv2)

    wscp = jnp.concatenate(
        [W_sc, jnp.zeros((S, 7, F), jnp.float32)], axis=1)
    z0f, z1p, ro = _node_call(
        ya, yb, nf0, nf1p, W_down_l0 * EPS, W_down_l1 * EPS,
        W_skip_l0, W_skip_l1, wscp, W_post_l0, W_post_l1, W_ro)

    z0 = z0f[:, :, None]
    z1 = jnp.transpose(z1p.reshape(n, 3, F), (0, 2, 1))
    return z0, z1, ro


# speedup vs baseline: 1.0697x; 1.0697x over previous
import functools
import numpy as np
import jax
import jax.numpy as jnp
from jax import lax
from jax.experimental import pallas as pl
from jax.experimental.pallas import tpu as pltpu
from jax.experimental.pallas import tpu_sc as plsc

F = 16
S = 10
EPS = 0.25
_SQ3 = float(np.sqrt(3.0))

_NC = 2
_NS = 16
_IDXW = 128
_CHUNK_ROWS = 8
_CE = _IDXW
_NPAD = 50048


def _up_body(nf0_ref, nf1_ref, w0_ref, w1_ref, out_ref):
    w0 = w0_ref[...]
    w1 = w1_ref[...]
    h0 = jnp.dot(nf0_ref[...], w0, preferred_element_type=jnp.float32)
    nf1 = nf1_ref[...]
    h1x = jnp.dot(nf1[:, 0:16], w1, preferred_element_type=jnp.float32)
    h1y = jnp.dot(nf1[:, 16:32], w1, preferred_element_type=jnp.float32)
    h1z = jnp.dot(nf1[:, 32:48], w1, preferred_element_type=jnp.float32)
    out_ref[...] = jnp.concatenate([h0, h1x, h1y, h1z], axis=1)


def _up_call(nf0, nf1p, W_up_l0, W_up_l1):
    n = nf0.shape[0]
    nb = 2000
    grid = n // nb
    return pl.pallas_call(
        _up_body,
        grid=(grid,),
        in_specs=[
            pl.BlockSpec((nb, F), lambda i: (i, 0)),
            pl.BlockSpec((nb, 3 * F), lambda i: (i, 0)),
            pl.BlockSpec((F, F), lambda i: (0, 0)),
            pl.BlockSpec((F, F), lambda i: (0, 0)),
        ],
        out_specs=pl.BlockSpec((nb, 4 * F), lambda i: (i, 0)),
        out_shape=jax.ShapeDtypeStruct((n, 4 * F), jnp.float32),
    )(nf0, nf1p, W_up_l0, W_up_l1)


def _silu(x):
    return x * jax.nn.sigmoid(x)


def _edge_body(nvalid, r_ref, m0_ref, m1_ref, m2_ref, m3_ref, out_ref):
    r = r_ref[...]
    h = _silu(jnp.dot(r, m0_ref[...], preferred_element_type=jnp.float32))
    h = _silu(jnp.dot(h, m1_ref[...], preferred_element_type=jnp.float32))
    h = _silu(jnp.dot(h, m2_ref[...], preferred_element_type=jnp.float32))
    mix = jnp.dot(h, m3_ref[...], preferred_element_type=jnp.float32)
    wscale = jnp.concatenate([
        jnp.ones((16,), jnp.float32),
        jnp.full((16,), 1.0 / _SQ3, jnp.float32),
        jnp.ones((48,), jnp.float32),
    ])[None, :]
    out_ref[...] = mix * wscale
    eb = r_ref.shape[0]
    base = pl.program_id(0) * eb

    @pl.when(base + eb > nvalid)
    def _():
        rid = base + jax.lax.broadcasted_iota(jnp.int32, (eb, 80), 0)
        out_ref[...] = jnp.where(rid < nvalid, out_ref[...], 0.0)


def _edge_call(r8, epad, mlp_w0, mlp_w1, mlp_w2, mlp_w3):
    e = r8.shape[0]
    eb = 2048
    grid = epad // eb
    last_in = (e - 1) // eb
    return pl.pallas_call(
        functools.partial(_edge_body, e),
        grid=(grid,),
        in_specs=[
            pl.BlockSpec((eb, 8), lambda i: (jnp.minimum(i, last_in), 0)),
            pl.BlockSpec(mlp_w0.shape, lambda i: (0, 0)),
            pl.BlockSpec(mlp_w1.shape, lambda i: (0, 0)),
            pl.BlockSpec(mlp_w2.shape, lambda i: (0, 0)),
            pl.BlockSpec(mlp_w3.shape, lambda i: (0, 0)),
        ],
        out_specs=pl.BlockSpec((eb, 80), lambda i: (i, 0)),
        out_shape=jax.ShapeDtypeStruct((epad, 80), jnp.float32),
    )(r8, mlp_w0, mlp_w1, mlp_w2, mlp_w3)


def _sh_body(v_ref, out_ref):
    x = v_ref[0:1, :]
    y = v_ref[1:2, :]
    z = v_ref[2:3, :]
    rn = jnp.sqrt(x * x + y * y + z * z)
    inv = 1.0 / (rn + 1e-9)
    ux = x * inv
    uy = y * inv
    uz = z * inv
    shx = _SQ3 * ux
    shy = _SQ3 * uy
    shz = _SQ3 * uz
    a = _SQ3 * ux * uy
    b = _SQ3 * uy * uz
    c = 1.5 * uz * uz - 0.5
    d = _SQ3 * ux * uz
    e = 0.5 * _SQ3 * (ux * ux - uy * uy)
    m00 = e - 0.5 * c
    m11 = -e - 0.5 * c
    zpad = jnp.zeros((7, x.shape[1]), jnp.float32)
    out_ref[...] = jnp.concatenate(
        [shx, shy, shz, m00, a, d, m11, b, c, zpad], axis=0)


def _sh_call(vecT):
    epad = vecT.shape[1]
    ebT = 16384
    grid = epad // ebT
    return pl.pallas_call(
        _sh_body,
        grid=(grid,),
        in_specs=[pl.BlockSpec((3, ebT), lambda i: (0, i))],
        out_specs=pl.BlockSpec((16, ebT), lambda i: (0, i)),
        out_shape=jax.ShapeDtypeStruct((16, epad), jnp.float32),
    )(vecT)


def _sc_body(h_hbm, p_hbm, s_hbm, snd_hbm, rcv_hbm, ya_hbm, yb_hbm,
             acc_sh, idx_s, idx_r, p_v, s_v, x_v, m_v, z_v,
             sem_p, sem_s, sem_g):
    core = lax.axis_index("c")
    sub = lax.axis_index("s")
    n = ya_hbm.shape[0]
    rows_per_sub_out = n // _NS
    zrows = z_v.shape[0]

    zero16 = jnp.zeros((16,), jnp.float32)

    def zinit(i, carry):
        z_v[i, 0:16] = zero16
        z_v[i, 16:32] = zero16
        return carry

    lax.fori_loop(0, zrows, zinit, 0)
    obase = sub * rows_per_sub_out

    def zcopy(i, carry):
        pltpu.sync_copy(z_v, acc_sh.at[pl.ds(obase + i * zrows, zrows)])
        return carry

    lax.fori_loop(0, rows_per_sub_out // zrows, zcopy, 0)
    plsc.subcore_barrier()

    total_rows = snd_hbm.shape[0]
    rows_per_sub = total_rows // _NS
    row0 = sub * rows_per_sub

    def compute_core0(e4, carry):
        for j in range(1):
            e = e4 + j
            x0 = x_v[e, 0:16]
            xx = x_v[e, 16:32]
            xy = x_v[e, 32:48]
            xz = x_v[e, 48:64]
            w0 = p_v[e, 0:16]
            w1 = p_v[e, 16:32]
            w2 = p_v[e, 32:48]
            w3 = p_v[e, 48:64]
            w4 = p_v[e, 64:80]
            sv = s_v[e, 0:16]
            shx = sv[0]
            shy = sv[1]
            shz = sv[2]
            m00 = sv[3]
            m01 = sv[4]
            m02 = sv[5]
            dot = xx * shx + xy * shy + xz * shz
            m0 = w0 * x0 + w1 * dot
            t = w2 * x0
            m1x = t * shx + w3 * xx + w4 * (xx * m00 + xy * m01 + xz * m02)
            m_v[e, 0:16] = m0
            m_v[e, 16:32] = m1x
        return carry

    def compute_core1(e4, carry):
        for j in range(1):
            e = e4 + j
            x0 = x_v[e, 0:16]
            xx = x_v[e, 16:32]
            xy = x_v[e, 32:48]
            xz = x_v[e, 48:64]
            w2 = p_v[e, 32:48]
            w3 = p_v[e, 48:64]
            w4 = p_v[e, 64:80]
            sv = s_v[e, 0:16]
            shy = sv[1]
            shz = sv[2]
            m01 = sv[4]
            m02 = sv[5]
            m11 = sv[6]
            m12 = sv[7]
            m22 = sv[8]
            t = w2 * x0
            m1y = t * shy + w3 * xy + w4 * (xx * m01 + xy * m11 + xz * m12)
            m1z = t * shz + w3 * xz + w4 * (xx * m02 + xy * m12 + xz * m22)
            m_v[e, 0:16] = m1y
            m_v[e, 16:32] = m1z
        return carry

    def chunk(k, carry):
        rbase = row0 + k * _CHUNK_ROWS
        pltpu.sync_copy(snd_hbm.at[pl.ds(rbase, _CHUNK_ROWS)], idx_s)
        pltpu.sync_copy(rcv_hbm.at[pl.ds(rbase, _CHUNK_ROWS)], idx_r)
        for h in range(_CHUNK_ROWS):
            cp = pltpu.async_copy(
                p_hbm.at[pl.ds((rbase + h) * _IDXW, _CE)], p_v, sem_p)
            cs = pltpu.async_copy(
                s_hbm.at[pl.ds((rbase + h) * _IDXW, _CE)], s_v, sem_s)
            gd = pltpu.async_copy(h_hbm.at[idx_s.at[h]], x_v, sem_g)
            cp.wait()
            cs.wait()
            gd.wait()

            @pl.when(core == 0)
            def _():
                lax.fori_loop(0, _CE, compute_core0, 0)

            @pl.when(core == 1)
            def _():
                lax.fori_loop(0, _CE, compute_core1, 0)

            pltpu.sync_copy(m_v, acc_sh.at[idx_r.at[h]], add=True)
        return carry

    lax.fori_loop(0, rows_per_sub // _CHUNK_ROWS, chunk, 0)
    plsc.subcore_barrier()

    @pl.when(core == 0)
    def _():
        pltpu.sync_copy(acc_sh.at[pl.ds(obase, rows_per_sub_out)],
                        ya_hbm.at[pl.ds(obase, rows_per_sub_out)])

    @pl.when(core == 1)
    def _():
        pltpu.sync_copy(acc_sh.at[pl.ds(obase, rows_per_sub_out)],
                        yb_hbm.at[pl.ds(obase, rows_per_sub_out)])


def _sc_call(h_tab, p_edge, s_edge, snd2, rcv2):
    mesh = plsc.VectorSubcoreMesh(
        core_axis_name="c", subcore_axis_name="s",
        num_cores=_NC, num_subcores=_NS)
    fn = functools.partial(
        pl.kernel,
        out_type=[
            jax.ShapeDtypeStruct((_NPAD, 32), jnp.float32),
            jax.ShapeDtypeStruct((_NPAD, 32), jnp.float32),
        ],
        mesh=mesh,
        scratch_types=[
            pltpu.VMEM_SHARED((_NPAD, 32), jnp.float32),
            pltpu.VMEM((_CHUNK_ROWS, _IDXW), jnp.int32),
            pltpu.VMEM((_CHUNK_ROWS, _IDXW), jnp.int32),
            pltpu.VMEM((_CE, 80), jnp.float32),
            pltpu.VMEM((_CE, 16), jnp.float32),
            pltpu.VMEM((_CE, 64), jnp.float32),
            pltpu.VMEM((_CE, 32), jnp.float32),
            pltpu.VMEM((8, 32), jnp.float32),
            pltpu.SemaphoreType.DMA,
            pltpu.SemaphoreType.DMA,
            pltpu.SemaphoreType.DMA,
        ],
        compiler_params=pltpu.CompilerParams(use_tc_tiling_on_sc=False),
    )(_sc_body)
    return fn(h_tab, p_edge, s_edge, snd2, rcv2)


def _node_body(ya_ref, yb_ref, nf0_ref, nf1_ref,
               wd0_ref, wd1_ref, wsk0_ref, wsk1_ref, wsc_ref,
               wp0_ref, wp1_ref, wro_ref,
               z0_ref, z1_ref, ro_ref):
    ya = ya_ref[...]
    yb = yb_ref[...]
    wd0 = wd0_ref[...]
    wd1 = wd1_ref[...]
    s = jnp.dot(ya[:, 0:16], wd0, preferred_element_type=jnp.float32)
    v1x = jnp.dot(ya[:, 16:32], wd1, preferred_element_type=jnp.float32)
    v1y = jnp.dot(yb[:, 0:16], wd1, preferred_element_type=jnp.float32)
    v1z = jnp.dot(yb[:, 16:32], wd1, preferred_element_type=jnp.float32)
    n2 = v1x * v1x + v1y * v1y + v1z * v1z
    wz = wsc_ref[0]
    s2 = s * s
    z0 = (wz[0:1, :] * s + wz[1:2, :] * s2 + wz[2:3, :] * (s2 * s)
          + wz[3:4, :] * n2 + wz[4:5, :] * (s * n2))
    t1 = wz[5:6, :] + wz[6:7, :] * s + wz[7:8, :] * s2 + wz[8:9, :] * n2
    z1x = t1 * v1x
    z1y = t1 * v1y
    z1z = t1 * v1z
    wp0 = wp0_ref[...]
    wp1 = wp1_ref[...]
    z0 = jnp.dot(z0, wp0, preferred_element_type=jnp.float32)
    z1x = jnp.dot(z1x, wp1, preferred_element_type=jnp.float32)
    z1y = jnp.dot(z1y, wp1, preferred_element_type=jnp.float32)
    z1z = jnp.dot(z1z, wp1, preferred_element_type=jnp.float32)
    nf1 = nf1_ref[...]
    wsk0 = wsk0_ref[0]
    wsk1 = wsk1_ref[0]
    z0 = z0 + jnp.dot(nf0_ref[...], wsk0, preferred_element_type=jnp.float32)
    z1x = z1x + jnp.dot(nf1[:, 0:16], wsk1, preferred_element_type=jnp.float32)
    z1y = z1y + jnp.dot(nf1[:, 16:32], wsk1,
                        preferred_element_type=jnp.float32)
    z1z = z1z + jnp.dot(nf1[:, 32:48], wsk1,
                        preferred_element_type=jnp.float32)
    z0_ref[...] = z0
    z1_ref[...] = jnp.concatenate([z1x, z1y, z1z], axis=1)
    ro_ref[...] = jnp.sum(z0 * wro_ref[...][:, 0][None, :], axis=1,
                          keepdims=True)


def _node_call(ya, yb, nf0, nf1p, wd0e, wd1e, wsk0, wsk1, wscp, wp0, wp1, wro):
    n = nf0.shape[0]
    nb = min(1000, n // S)
    bps = (n // S) // nb
    grid = n // nb

    return pl.pallas_call(
        _node_body,
        grid=(grid,),
        in_specs=[
            pl.BlockSpec((nb, 32), lambda i: (i, 0)),
            pl.BlockSpec((nb, 32), lambda i: (i, 0)),
            pl.BlockSpec((nb, F), lambda i: (i, 0)),
            pl.BlockSpec((nb, 3 * F), lambda i: (i, 0)),
            pl.BlockSpec((F, F), lambda i: (0, 0)),
            pl.BlockSpec((F, F), lambda i: (0, 0)),
            pl.BlockSpec((1, F, F), lambda i: (i // bps, 0, 0)),
            pl.BlockSpec((1, F, F), lambda i: (i // bps, 0, 0)),
            pl.BlockSpec((1, F, F), lambda i: (i // bps, 0, 0)),
            pl.BlockSpec((F, F), lambda i: (0, 0)),
            pl.BlockSpec((F, F), lambda i: (0, 0)),
            pl.BlockSpec((F, 1), lambda i: (0, 0)),
        ],
        out_specs=[
            pl.BlockSpec((nb, F), lambda i: (i, 0)),
            pl.BlockSpec((nb, 3 * F), lambda i: (i, 0)),
            pl.BlockSpec((nb, 1), lambda i: (i, 0)),
        ],
        out_shape=[
            jax.ShapeDtypeStruct((n, F), jnp.float32),
            jax.ShapeDtypeStruct((n, 3 * F), jnp.float32),
            jax.ShapeDtypeStruct((n, 1), jnp.float32),
        ],
    )(ya, yb, nf0, nf1p, wd0e, wd1e, wsk0, wsk1, wscp, wp0, wp1, wro)


def kernel(vectors, node_feats_l0, node_feats_l1, num_species_counts,
           radial_embeddings, senders, receivers, num_nodes,
           W_up_l0, W_up_l1, mlp_w0, mlp_w1, mlp_w2, mlp_w3,
           W_down_l0, W_down_l1, W_skip_l0, W_skip_l1, W_sc,
           W_post_l0, W_post_l1, W_ro):
    n = node_feats_l0.shape[0]
    e = vectors.shape[0]
    epb = _IDXW * _NS * _CHUNK_ROWS
    e_pad = ((e + epb - 1) // epb) * epb

    nf0 = node_feats_l0[:, :, 0]
    nf1p = jnp.transpose(node_feats_l1, (0, 2, 1)).reshape(n, 3 * F)

    h_tab = _up_call(nf0, nf1p, W_up_l0, W_up_l1)

    p_edge = _edge_call(radial_embeddings, e_pad,
                        mlp_w0, mlp_w1, mlp_w2, mlp_w3)
    vecT = jnp.pad(vectors.T, ((0, 0), (0, e_pad - e)))
    s_edge = _sh_call(vecT).T

    zpad_i = jnp.zeros((e_pad - e,), jnp.int32)
    snd2 = jnp.concatenate([senders.astype(jnp.int32), zpad_i]).reshape(
        e_pad // _IDXW, _IDXW)
    rcv2 = jnp.concatenate([receivers.astype(jnp.int32), zpad_i]).reshape(
        e_pad // _IDXW, _IDXW)

    ya, yb = _sc_call(h_tab, p_edge, s_edge, snd2, rcv2)

    wscp = jnp.concatenate(
        [W_sc, jnp.zeros((S, 7, F), jnp.float32)], axis=1)
    z0f, z1p, ro = _node_call(
        ya, yb, nf0, nf1p, W_down_l0 * EPS, W_down_l1 * EPS,
        W_skip_l0, W_skip_l1, wscp, W_post_l0, W_post_l1, W_ro)

    z0 = z0f[:, :, None]
    z1 = jnp.transpose(z1p.reshape(n, 3, F), (0, 2, 1))
    return z0, z1, ro


# speedup vs baseline: 1.2814x; 1.1979x over previous
import functools
import numpy as np
import jax
import jax.numpy as jnp
from jax import lax
from jax.experimental import pallas as pl
from jax.experimental.pallas import tpu as pltpu
from jax.experimental.pallas import tpu_sc as plsc

F = 16
S = 10
EPS = 0.25
_SQ3 = float(np.sqrt(3.0))

_NC = 2
_NS = 16
_IDXW = 128
_CHUNK_ROWS = 8
_CE = _IDXW
_NPAD = 50048


def _up_body(nf0_ref, nf1_ref, w0_ref, w1_ref, out_ref):
    w0 = w0_ref[...]
    w1 = w1_ref[...]
    h0 = jnp.dot(nf0_ref[...], w0, preferred_element_type=jnp.float32)
    nf1 = nf1_ref[...]
    h1x = jnp.dot(nf1[:, 0:16], w1, preferred_element_type=jnp.float32)
    h1y = jnp.dot(nf1[:, 16:32], w1, preferred_element_type=jnp.float32)
    h1z = jnp.dot(nf1[:, 32:48], w1, preferred_element_type=jnp.float32)
    out_ref[...] = jnp.concatenate([h0, h1x, h1y, h1z], axis=1)


def _up_call(nf0, nf1p, W_up_l0, W_up_l1):
    n = nf0.shape[0]
    nb = 2000
    grid = n // nb
    return pl.pallas_call(
        _up_body,
        grid=(grid,),
        in_specs=[
            pl.BlockSpec((nb, F), lambda i: (i, 0)),
            pl.BlockSpec((nb, 3 * F), lambda i: (i, 0)),
            pl.BlockSpec((F, F), lambda i: (0, 0)),
            pl.BlockSpec((F, F), lambda i: (0, 0)),
        ],
        out_specs=pl.BlockSpec((nb, 4 * F), lambda i: (i, 0)),
        out_shape=jax.ShapeDtypeStruct((n, 4 * F), jnp.float32),
    )(nf0, nf1p, W_up_l0, W_up_l1)


def _silu(x):
    return x * jax.nn.sigmoid(x)


def _edge_body(nvalid, r_ref, m0_ref, m1_ref, m2_ref, m3_ref, out_ref):
    r = r_ref[...]
    h = _silu(jnp.dot(r, m0_ref[...], preferred_element_type=jnp.float32))
    h = _silu(jnp.dot(h, m1_ref[...], preferred_element_type=jnp.float32))
    h = _silu(jnp.dot(h, m2_ref[...], preferred_element_type=jnp.float32))
    mix = jnp.dot(h, m3_ref[...], preferred_element_type=jnp.float32)
    wscale = jnp.concatenate([
        jnp.ones((16,), jnp.float32),
        jnp.full((16,), 1.0 / _SQ3, jnp.float32),
        jnp.ones((48,), jnp.float32),
    ])[None, :]
    out_ref[...] = mix * wscale
    eb = r_ref.shape[0]
    base = pl.program_id(0) * eb

    @pl.when(base + eb > nvalid)
    def _():
        rid = base + jax.lax.broadcasted_iota(jnp.int32, (eb, 80), 0)
        out_ref[...] = jnp.where(rid < nvalid, out_ref[...], 0.0)


def _edge_call(r8, epad, mlp_w0, mlp_w1, mlp_w2, mlp_w3):
    e = r8.shape[0]
    eb = 2048
    grid = epad // eb
    last_in = (e - 1) // eb
    return pl.pallas_call(
        functools.partial(_edge_body, e),
        grid=(grid,),
        in_specs=[
            pl.BlockSpec((eb, 8), lambda i: (jnp.minimum(i, last_in), 0)),
            pl.BlockSpec(mlp_w0.shape, lambda i: (0, 0)),
            pl.BlockSpec(mlp_w1.shape, lambda i: (0, 0)),
            pl.BlockSpec(mlp_w2.shape, lambda i: (0, 0)),
            pl.BlockSpec(mlp_w3.shape, lambda i: (0, 0)),
        ],
        out_specs=pl.BlockSpec((eb, 80), lambda i: (i, 0)),
        out_shape=jax.ShapeDtypeStruct((epad, 80), jnp.float32),
    )(r8, mlp_w0, mlp_w1, mlp_w2, mlp_w3)


def _sh_body(v_ref, out_ref):
    x = v_ref[0:1, :]
    y = v_ref[1:2, :]
    z = v_ref[2:3, :]
    rn = jnp.sqrt(x * x + y * y + z * z)
    inv = 1.0 / (rn + 1e-9)
    ux = x * inv
    uy = y * inv
    uz = z * inv
    shx = _SQ3 * ux
    shy = _SQ3 * uy
    shz = _SQ3 * uz
    a = _SQ3 * ux * uy
    b = _SQ3 * uy * uz
    c = 1.5 * uz * uz - 0.5
    d = _SQ3 * ux * uz
    e = 0.5 * _SQ3 * (ux * ux - uy * uy)
    m00 = e - 0.5 * c
    m11 = -e - 0.5 * c
    zpad = jnp.zeros((7, x.shape[1]), jnp.float32)
    out_ref[...] = jnp.concatenate(
        [shx, shy, shz, m00, a, d, m11, b, c, zpad], axis=0)


def _sh_call(vecT):
    epad = vecT.shape[1]
    ebT = 16384
    grid = epad // ebT
    return pl.pallas_call(
        _sh_body,
        grid=(grid,),
        in_specs=[pl.BlockSpec((3, ebT), lambda i: (0, i))],
        out_specs=pl.BlockSpec((16, ebT), lambda i: (0, i)),
        out_shape=jax.ShapeDtypeStruct((16, epad), jnp.float32),
    )(vecT)


def _sc_body(h_hbm, p_hbm, s_hbm, snd_hbm, rcv_hbm, ya_hbm, yb_hbm,
             acc_sh, idx_s, idx_r, p_v, s_v, x_v, m_v, z_v,
             sem_p, sem_s, sem_g):
    core = lax.axis_index("c")
    sub = lax.axis_index("s")
    n = ya_hbm.shape[0]
    rows_per_sub_out = n // _NS
    zrows = z_v.shape[0]

    zero16 = jnp.zeros((16,), jnp.float32)

    def zinit(i, carry):
        z_v[i, 0:16] = zero16
        z_v[i, 16:32] = zero16
        return carry

    lax.fori_loop(0, zrows, zinit, 0)
    obase = sub * rows_per_sub_out

    def zcopy(i, carry):
        pltpu.sync_copy(z_v, acc_sh.at[pl.ds(obase + i * zrows, zrows)])
        return carry

    lax.fori_loop(0, rows_per_sub_out // zrows, zcopy, 0)
    plsc.subcore_barrier()

    total_rows = snd_hbm.shape[0]
    rows_per_sub = total_rows // _NS
    row0 = sub * rows_per_sub

    def compute_core0(e4, carry):
        for j in range(1):
            e = e4 + j
            x0 = x_v[e, 0:16]
            xx = x_v[e, 16:32]
            xy = x_v[e, 32:48]
            xz = x_v[e, 48:64]
            w0 = p_v[e, 0:16]
            w1 = p_v[e, 16:32]
            w2 = p_v[e, 32:48]
            w3 = p_v[e, 48:64]
            w4 = p_v[e, 64:80]
            sv = s_v[e, 0:16]
            shx = sv[0]
            shy = sv[1]
            shz = sv[2]
            m00 = sv[3]
            m01 = sv[4]
            m02 = sv[5]
            dot = xx * shx + xy * shy + xz * shz
            m0 = w0 * x0 + w1 * dot
            t = w2 * x0
            m1x = t * shx + w3 * xx + w4 * (xx * m00 + xy * m01 + xz * m02)
            m_v[e, 0:16] = m0
            m_v[e, 16:32] = m1x
        return carry

    def compute_core1(e4, carry):
        for j in range(1):
            e = e4 + j
            x0 = x_v[e, 0:16]
            xx = x_v[e, 16:32]
            xy = x_v[e, 32:48]
            xz = x_v[e, 48:64]
            w2 = p_v[e, 32:48]
            w3 = p_v[e, 48:64]
            w4 = p_v[e, 64:80]
            sv = s_v[e, 0:16]
            shy = sv[1]
            shz = sv[2]
            m01 = sv[4]
            m02 = sv[5]
            m11 = sv[6]
            m12 = sv[7]
            m22 = sv[8]
            t = w2 * x0
            m1y = t * shy + w3 * xy + w4 * (xx * m01 + xy * m11 + xz * m12)
            m1z = t * shz + w3 * xz + w4 * (xx * m02 + xy * m12 + xz * m22)
            m_v[e, 0:16] = m1y
            m_v[e, 16:32] = m1z
        return carry

    def chunk(k, carry):
        rbase = row0 + k * _CHUNK_ROWS
        pltpu.sync_copy(snd_hbm.at[pl.ds(rbase, _CHUNK_ROWS)], idx_s)
        pltpu.sync_copy(rcv_hbm.at[pl.ds(rbase, _CHUNK_ROWS)], idx_r)
        for h in range(_CHUNK_ROWS):
            cp = pltpu.async_copy(
                p_hbm.at[pl.ds((rbase + h) * _IDXW, _CE)], p_v, sem_p)
            cs = pltpu.async_copy(
                s_hbm.at[pl.ds((rbase + h) * _IDXW, _CE)], s_v, sem_s)
            gd = pltpu.async_copy(h_hbm.at[idx_s.at[h]], x_v, sem_g)
            cp.wait()
            cs.wait()
            gd.wait()

            @pl.when(core == 0)
            def _():
                lax.fori_loop(0, _CE, compute_core0, 0)

            @pl.when(core == 1)
            def _():
                lax.fori_loop(0, _CE, compute_core1, 0)

            pltpu.sync_copy(m_v, acc_sh.at[idx_r.at[h]], add=True)
        return carry

    lax.fori_loop(0, rows_per_sub // _CHUNK_ROWS, chunk, 0)
    plsc.subcore_barrier()

    @pl.when(core == 0)
    def _():
        pltpu.sync_copy(acc_sh.at[pl.ds(obase, rows_per_sub_out)],
                        ya_hbm.at[pl.ds(obase, rows_per_sub_out)])

    @pl.when(core == 1)
    def _():
        pltpu.sync_copy(acc_sh.at[pl.ds(obase, rows_per_sub_out)],
                        yb_hbm.at[pl.ds(obase, rows_per_sub_out)])


def _sc_call(h_tab, p_edge, s_edge, snd2, rcv2):
    mesh = plsc.VectorSubcoreMesh(
        core_axis_name="c", subcore_axis_name="s",
        num_cores=_NC, num_subcores=_NS)
    fn = functools.partial(
        pl.kernel,
        out_type=[
            jax.ShapeDtypeStruct((_NPAD, 32), jnp.float32),
            jax.ShapeDtypeStruct((_NPAD, 32), jnp.float32),
        ],
        mesh=mesh,
        scratch_types=[
            pltpu.VMEM_SHARED((_NPAD, 32), jnp.float32),
            pltpu.VMEM((_CHUNK_ROWS, _IDXW), jnp.int32),
            pltpu.VMEM((_CHUNK_ROWS, _IDXW), jnp.int32),
            pltpu.VMEM((_CE, 80), jnp.float32),
            pltpu.VMEM((_CE, 16), jnp.float32),
            pltpu.VMEM((_CE, 64), jnp.float32),
            pltpu.VMEM((_CE, 32), jnp.float32),
            pltpu.VMEM((8, 32), jnp.float32),
            pltpu.SemaphoreType.DMA,
            pltpu.SemaphoreType.DMA,
            pltpu.SemaphoreType.DMA,
        ],
        compiler_params=pltpu.CompilerParams(use_tc_tiling_on_sc=False),
    )(_sc_body)
    return fn(h_tab, p_edge, s_edge, snd2, rcv2)


def _node_body(ya_ref, yb_ref, ya2_ref, yb2_ref, nf0_ref, nf1_ref,
               wd0_ref, wd1_ref, wsk0_ref, wsk1_ref, wsc_ref,
               wp0_ref, wp1_ref, wro_ref,
               z0_ref, z1_ref, ro_ref):
    ya = ya_ref[...] + ya2_ref[...]
    yb = yb_ref[...] + yb2_ref[...]
    wd0 = wd0_ref[...]
    wd1 = wd1_ref[...]
    s = jnp.dot(ya[:, 0:16], wd0, preferred_element_type=jnp.float32)
    v1x = jnp.dot(ya[:, 16:32], wd1, preferred_element_type=jnp.float32)
    v1y = jnp.dot(yb[:, 0:16], wd1, preferred_element_type=jnp.float32)
    v1z = jnp.dot(yb[:, 16:32], wd1, preferred_element_type=jnp.float32)
    n2 = v1x * v1x + v1y * v1y + v1z * v1z
    wz = wsc_ref[0]
    s2 = s * s
    z0 = (wz[0:1, :] * s + wz[1:2, :] * s2 + wz[2:3, :] * (s2 * s)
          + wz[3:4, :] * n2 + wz[4:5, :] * (s * n2))
    t1 = wz[5:6, :] + wz[6:7, :] * s + wz[7:8, :] * s2 + wz[8:9, :] * n2
    z1x = t1 * v1x
    z1y = t1 * v1y
    z1z = t1 * v1z
    wp0 = wp0_ref[...]
    wp1 = wp1_ref[...]
    z0 = jnp.dot(z0, wp0, preferred_element_type=jnp.float32)
    z1x = jnp.dot(z1x, wp1, preferred_element_type=jnp.float32)
    z1y = jnp.dot(z1y, wp1, preferred_element_type=jnp.float32)
    z1z = jnp.dot(z1z, wp1, preferred_element_type=jnp.float32)
    nf1 = nf1_ref[...]
    wsk0 = wsk0_ref[0]
    wsk1 = wsk1_ref[0]
    z0 = z0 + jnp.dot(nf0_ref[...], wsk0, preferred_element_type=jnp.float32)
    z1x = z1x + jnp.dot(nf1[:, 0:16], wsk1, preferred_element_type=jnp.float32)
    z1y = z1y + jnp.dot(nf1[:, 16:32], wsk1,
                        preferred_element_type=jnp.float32)
    z1z = z1z + jnp.dot(nf1[:, 32:48], wsk1,
                        preferred_element_type=jnp.float32)
    z0_ref[...] = z0
    z1_ref[...] = jnp.concatenate([z1x, z1y, z1z], axis=1)
    ro_ref[...] = jnp.sum(z0 * wro_ref[...][:, 0][None, :], axis=1,
                          keepdims=True)


def _node_call(ya, yb, ya2, yb2, nf0, nf1p,
               wd0e, wd1e, wsk0, wsk1, wscp, wp0, wp1, wro):
    n = nf0.shape[0]
    nb = min(1000, n // S)
    bps = (n // S) // nb
    grid = n // nb

    return pl.pallas_call(
        _node_body,
        grid=(grid,),
        in_specs=[
            pl.BlockSpec((nb, 32), lambda i: (i, 0)),
            pl.BlockSpec((nb, 32), lambda i: (i, 0)),
            pl.BlockSpec((nb, 32), lambda i: (i, 0)),
            pl.BlockSpec((nb, 32), lambda i: (i, 0)),
            pl.BlockSpec((nb, F), lambda i: (i, 0)),
            pl.BlockSpec((nb, 3 * F), lambda i: (i, 0)),
            pl.BlockSpec((F, F), lambda i: (0, 0)),
            pl.BlockSpec((F, F), lambda i: (0, 0)),
            pl.BlockSpec((1, F, F), lambda i: (i // bps, 0, 0)),
            pl.BlockSpec((1, F, F), lambda i: (i // bps, 0, 0)),
            pl.BlockSpec((1, F, F), lambda i: (i // bps, 0, 0)),
            pl.BlockSpec((F, F), lambda i: (0, 0)),
            pl.BlockSpec((F, F), lambda i: (0, 0)),
            pl.BlockSpec((F, 1), lambda i: (0, 0)),
        ],
        out_specs=[
            pl.BlockSpec((nb, F), lambda i: (i, 0)),
            pl.BlockSpec((nb, 3 * F), lambda i: (i, 0)),
            pl.BlockSpec((nb, 1), lambda i: (i, 0)),
        ],
        out_shape=[
            jax.ShapeDtypeStruct((n, F), jnp.float32),
            jax.ShapeDtypeStruct((n, 3 * F), jnp.float32),
            jax.ShapeDtypeStruct((n, 1), jnp.float32),
        ],
    )(ya, yb, ya2, yb2, nf0, nf1p,
      wd0e, wd1e, wsk0, wsk1, wscp, wp0, wp1, wro)


def kernel(vectors, node_feats_l0, node_feats_l1, num_species_counts,
           radial_embeddings, senders, receivers, num_nodes,
           W_up_l0, W_up_l1, mlp_w0, mlp_w1, mlp_w2, mlp_w3,
           W_down_l0, W_down_l1, W_skip_l0, W_skip_l1, W_sc,
           W_post_l0, W_post_l1, W_ro):
    n = node_feats_l0.shape[0]
    e = vectors.shape[0]
    epb = _IDXW * _NS * _CHUNK_ROWS
    e_pad = ((e + epb - 1) // epb) * epb

    nf0 = node_feats_l0[:, :, 0]
    nf1p = jnp.transpose(node_feats_l1, (0, 2, 1)).reshape(n, 3 * F)

    h_tab = _up_call(nf0, nf1p, W_up_l0, W_up_l1)

    h1 = ((e_pad // epb + 1) // 2) * epb
    vecT = jnp.pad(vectors.T, ((0, 0), (0, e_pad - e)))

    zpad_i = jnp.zeros((e_pad - e,), jnp.int32)
    snd2 = jnp.concatenate([senders.astype(jnp.int32), zpad_i]).reshape(
        e_pad // _IDXW, _IDXW)
    rcv2 = jnp.concatenate([receivers.astype(jnp.int32), zpad_i]).reshape(
        e_pad // _IDXW, _IDXW)
    r1 = h1 // _IDXW

    p1 = _edge_call(radial_embeddings[:h1], h1,
                    mlp_w0, mlp_w1, mlp_w2, mlp_w3)
    s1 = _sh_call(vecT[:, :h1]).T
    ya1, yb1 = _sc_call(h_tab, p1, s1, snd2[:r1], rcv2[:r1])

    p2 = _edge_call(radial_embeddings[h1:], e_pad - h1,
                    mlp_w0, mlp_w1, mlp_w2, mlp_w3)
    s2 = _sh_call(vecT[:, h1:]).T
    ya2, yb2 = _sc_call(h_tab, p2, s2, snd2[r1:], rcv2[r1:])

    wscp = jnp.concatenate(
        [W_sc, jnp.zeros((S, 7, F), jnp.float32)], axis=1)
    z0f, z1p, ro = _node_call(
        ya1, yb1, ya2, yb2, nf0, nf1p, W_down_l0 * EPS, W_down_l1 * EPS,
        W_skip_l0, W_skip_l1, wscp, W_post_l0, W_post_l1, W_ro)

    z0 = z0f[:, :, None]
    z1 = jnp.transpose(z1p.reshape(n, 3, F), (0, 2, 1))
    return z0, z1, ro


# speedup vs baseline: 1.3562x; 1.0583x over previous
import functools
import numpy as np
import jax
import jax.numpy as jnp
from jax import lax
from jax.experimental import pallas as pl
from jax.experimental.pallas import tpu as pltpu
from jax.experimental.pallas import tpu_sc as plsc

F = 16
S = 10
EPS = 0.25
_SQ3 = float(np.sqrt(3.0))

_NC = 2
_NS = 16
_IDXW = 128
_CHUNK_ROWS = 8
_CE = _IDXW
_NPAD = 50048


def _up_body(nf0_ref, nf1_ref, w0_ref, w1_ref, out_ref):
    w0 = w0_ref[...]
    w1 = w1_ref[...]
    h0 = jnp.dot(nf0_ref[...], w0, preferred_element_type=jnp.float32)
    nf1 = nf1_ref[...]
    h1x = jnp.dot(nf1[:, 0:16], w1, preferred_element_type=jnp.float32)
    h1y = jnp.dot(nf1[:, 16:32], w1, preferred_element_type=jnp.float32)
    h1z = jnp.dot(nf1[:, 32:48], w1, preferred_element_type=jnp.float32)
    out_ref[...] = jnp.concatenate([h0, h1x, h1y, h1z], axis=1)


def _up_call(nf0, nf1p, W_up_l0, W_up_l1):
    n = nf0.shape[0]
    nb = 2000
    grid = n // nb
    return pl.pallas_call(
        _up_body,
        grid=(grid,),
        in_specs=[
            pl.BlockSpec((nb, F), lambda i: (i, 0)),
            pl.BlockSpec((nb, 3 * F), lambda i: (i, 0)),
            pl.BlockSpec((F, F), lambda i: (0, 0)),
            pl.BlockSpec((F, F), lambda i: (0, 0)),
        ],
        out_specs=pl.BlockSpec((nb, 4 * F), lambda i: (i, 0)),
        out_shape=jax.ShapeDtypeStruct((n, 4 * F), jnp.float32),
    )(nf0, nf1p, W_up_l0, W_up_l1)


def _silu(x):
    return x * jax.nn.sigmoid(x)


def _edge_body(nvalid, r_ref, m0_ref, m1_ref, m2_ref, m3_ref, out_ref):
    r = r_ref[...]
    h = _silu(jnp.dot(r, m0_ref[...], preferred_element_type=jnp.float32))
    h = _silu(jnp.dot(h, m1_ref[...], preferred_element_type=jnp.float32))
    h = _silu(jnp.dot(h, m2_ref[...], preferred_element_type=jnp.float32))
    mix = jnp.dot(h, m3_ref[...], preferred_element_type=jnp.float32)
    wscale = jnp.concatenate([
        jnp.ones((16,), jnp.float32),
        jnp.full((16,), 1.0 / _SQ3, jnp.float32),
        jnp.ones((48,), jnp.float32),
    ])[None, :]
    out_ref[...] = mix * wscale
    eb = r_ref.shape[0]
    base = pl.program_id(0) * eb

    @pl.when(base + eb > nvalid)
    def _():
        rid = base + jax.lax.broadcasted_iota(jnp.int32, (eb, 80), 0)
        out_ref[...] = jnp.where(rid < nvalid, out_ref[...], 0.0)


def _edge_call(r8, epad, mlp_w0, mlp_w1, mlp_w2, mlp_w3):
    e = r8.shape[0]
    eb = 4096
    grid = epad // eb
    last_in = (e - 1) // eb
    return pl.pallas_call(
        functools.partial(_edge_body, e),
        grid=(grid,),
        in_specs=[
            pl.BlockSpec((eb, 8), lambda i: (jnp.minimum(i, last_in), 0)),
            pl.BlockSpec(mlp_w0.shape, lambda i: (0, 0)),
            pl.BlockSpec(mlp_w1.shape, lambda i: (0, 0)),
            pl.BlockSpec(mlp_w2.shape, lambda i: (0, 0)),
            pl.BlockSpec(mlp_w3.shape, lambda i: (0, 0)),
        ],
        out_specs=pl.BlockSpec((eb, 80), lambda i: (i, 0)),
        out_shape=jax.ShapeDtypeStruct((epad, 80), jnp.float32),
    )(r8, mlp_w0, mlp_w1, mlp_w2, mlp_w3)


def _sh_body(v_ref, out_ref):
    x = v_ref[0:1, :]
    y = v_ref[1:2, :]
    z = v_ref[2:3, :]
    rn = jnp.sqrt(x * x + y * y + z * z)
    inv = 1.0 / (rn + 1e-9)
    ux = x * inv
    uy = y * inv
    uz = z * inv
    shx = _SQ3 * ux
    shy = _SQ3 * uy
    shz = _SQ3 * uz
    a = _SQ3 * ux * uy
    b = _SQ3 * uy * uz
    c = 1.5 * uz * uz - 0.5
    d = _SQ3 * ux * uz
    e = 0.5 * _SQ3 * (ux * ux - uy * uy)
    m00 = e - 0.5 * c
    m11 = -e - 0.5 * c
    zpad = jnp.zeros((7, x.shape[1]), jnp.float32)
    out_ref[...] = jnp.concatenate(
        [shx, shy, shz, m00, a, d, m11, b, c, zpad], axis=0)


def _sh_call(vecT):
    epad = vecT.shape[1]
    ebT = 16384
    grid = epad // ebT
    return pl.pallas_call(
        _sh_body,
        grid=(grid,),
        in_specs=[pl.BlockSpec((3, ebT), lambda i: (0, i))],
        out_specs=pl.BlockSpec((16, ebT), lambda i: (0, i)),
        out_shape=jax.ShapeDtypeStruct((16, epad), jnp.float32),
    )(vecT)


def _sc_body(chained, h_hbm, p_hbm, s_hbm, snd_hbm, rcv_hbm, *rest):
    if chained:
        (yain_hbm, ybin_hbm, ya_hbm, yb_hbm,
         acc_sh, idx_s, idx_r, p_v, s_v, x_v, m_v, z_v,
         sem_p, sem_s, sem_g) = rest
    else:
        (ya_hbm, yb_hbm,
         acc_sh, idx_s, idx_r, p_v, s_v, x_v, m_v, z_v,
         sem_p, sem_s, sem_g) = rest
    core = lax.axis_index("c")
    sub = lax.axis_index("s")
    n = ya_hbm.shape[0]
    rows_per_sub_out = n // _NS
    zrows = z_v.shape[0]
    obase = sub * rows_per_sub_out

    if chained:
        @pl.when(core == 0)
        def _():
            pltpu.sync_copy(yain_hbm.at[pl.ds(obase, rows_per_sub_out)],
                            acc_sh.at[pl.ds(obase, rows_per_sub_out)])

        @pl.when(core == 1)
        def _():
            pltpu.sync_copy(ybin_hbm.at[pl.ds(obase, rows_per_sub_out)],
                            acc_sh.at[pl.ds(obase, rows_per_sub_out)])
    else:
        zero16 = jnp.zeros((16,), jnp.float32)

        def zinit(i, carry):
            z_v[i, 0:16] = zero16
            z_v[i, 16:32] = zero16
            return carry

        lax.fori_loop(0, zrows, zinit, 0)

        def zcopy(i, carry):
            pltpu.sync_copy(z_v, acc_sh.at[pl.ds(obase + i * zrows, zrows)])
            return carry

        lax.fori_loop(0, rows_per_sub_out // zrows, zcopy, 0)
    plsc.subcore_barrier()

    total_rows = snd_hbm.shape[0]
    rows_per_sub = total_rows // _NS
    row0 = sub * rows_per_sub

    def compute_core0(e4, carry):
        for j in range(1):
            e = e4 + j
            x0 = x_v[e, 0:16]
            xx = x_v[e, 16:32]
            xy = x_v[e, 32:48]
            xz = x_v[e, 48:64]
            w0 = p_v[e, 0:16]
            w1 = p_v[e, 16:32]
            w2 = p_v[e, 32:48]
            w3 = p_v[e, 48:64]
            w4 = p_v[e, 64:80]
            sv = s_v[e, 0:16]
            shx = sv[0]
            shy = sv[1]
            shz = sv[2]
            m00 = sv[3]
            m01 = sv[4]
            m02 = sv[5]
            dot = xx * shx + xy * shy + xz * shz
            m0 = w0 * x0 + w1 * dot
            t = w2 * x0
            m1x = t * shx + w3 * xx + w4 * (xx * m00 + xy * m01 + xz * m02)
            m_v[e, 0:16] = m0
            m_v[e, 16:32] = m1x
        return carry

    def compute_core1(e4, carry):
        for j in range(1):
            e = e4 + j
            x0 = x_v[e, 0:16]
            xx = x_v[e, 16:32]
            xy = x_v[e, 32:48]
            xz = x_v[e, 48:64]
            w2 = p_v[e, 32:48]
            w3 = p_v[e, 48:64]
            w4 = p_v[e, 64:80]
            sv = s_v[e, 0:16]
            shy = sv[1]
            shz = sv[2]
            m01 = sv[4]
            m02 = sv[5]
            m11 = sv[6]
            m12 = sv[7]
            m22 = sv[8]
            t = w2 * x0
            m1y = t * shy + w3 * xy + w4 * (xx * m01 + xy * m11 + xz * m12)
            m1z = t * shz + w3 * xz + w4 * (xx * m02 + xy * m12 + xz * m22)
            m_v[e, 0:16] = m1y
            m_v[e, 16:32] = m1z
        return carry

    def chunk(k, carry):
        rbase = row0 + k * _CHUNK_ROWS
        pltpu.sync_copy(snd_hbm.at[pl.ds(rbase, _CHUNK_ROWS)], idx_s)
        pltpu.sync_copy(rcv_hbm.at[pl.ds(rbase, _CHUNK_ROWS)], idx_r)
        for h in range(_CHUNK_ROWS):
            cp = pltpu.async_copy(
                p_hbm.at[pl.ds((rbase + h) * _IDXW, _CE)], p_v, sem_p)
            cs = pltpu.async_copy(
                s_hbm.at[pl.ds((rbase + h) * _IDXW, _CE)], s_v, sem_s)
            gd = pltpu.async_copy(h_hbm.at[idx_s.at[h]], x_v, sem_g)
            cp.wait()
            cs.wait()
            gd.wait()

            @pl.when(core == 0)
            def _():
                lax.fori_loop(0, _CE, compute_core0, 0)

            @pl.when(core == 1)
            def _():
                lax.fori_loop(0, _CE, compute_core1, 0)

            pltpu.sync_copy(m_v, acc_sh.at[idx_r.at[h]], add=True)
        return carry

    lax.fori_loop(0, rows_per_sub // _CHUNK_ROWS, chunk, 0)
    plsc.subcore_barrier()

    @pl.when(core == 0)
    def _():
        pltpu.sync_copy(acc_sh.at[pl.ds(obase, rows_per_sub_out)],
                        ya_hbm.at[pl.ds(obase, rows_per_sub_out)])

    @pl.when(core == 1)
    def _():
        pltpu.sync_copy(acc_sh.at[pl.ds(obase, rows_per_sub_out)],
                        yb_hbm.at[pl.ds(obase, rows_per_sub_out)])


def _sc_call(h_tab, p_edge, s_edge, snd2, rcv2, ya_in=None, yb_in=None):
    mesh = plsc.VectorSubcoreMesh(
        core_axis_name="c", subcore_axis_name="s",
        num_cores=_NC, num_subcores=_NS)
    chained = ya_in is not None
    fn = functools.partial(
        pl.kernel,
        out_type=[
            jax.ShapeDtypeStruct((_NPAD, 32), jnp.float32),
            jax.ShapeDtypeStruct((_NPAD, 32), jnp.float32),
        ],
        mesh=mesh,
        scratch_types=[
            pltpu.VMEM_SHARED((_NPAD, 32), jnp.float32),
            pltpu.VMEM((_CHUNK_ROWS, _IDXW), jnp.int32),
            pltpu.VMEM((_CHUNK_ROWS, _IDXW), jnp.int32),
            pltpu.VMEM((_CE, 80), jnp.float32),
            pltpu.VMEM((_CE, 16), jnp.float32),
            pltpu.VMEM((_CE, 64), jnp.float32),
            pltpu.VMEM((_CE, 32), jnp.float32),
            pltpu.VMEM((8, 32), jnp.float32),
            pltpu.SemaphoreType.DMA,
            pltpu.SemaphoreType.DMA,
            pltpu.SemaphoreType.DMA,
        ],
        compiler_params=pltpu.CompilerParams(use_tc_tiling_on_sc=False),
    )(functools.partial(_sc_body, chained))
    if chained:
        return fn(h_tab, p_edge, s_edge, snd2, rcv2, ya_in, yb_in)
    return fn(h_tab, p_edge, s_edge, snd2, rcv2)


def _node_body(ya_ref, yb_ref, nf0_ref, nf1_ref,
               wd0_ref, wd1_ref, wsk0_ref, wsk1_ref, wsc_ref,
               wp0_ref, wp1_ref, wro_ref,
               z0_ref, z1_ref, ro_ref):
    ya = ya_ref[...]
    yb = yb_ref[...]
    wd0 = wd0_ref[...]
    wd1 = wd1_ref[...]
    s = jnp.dot(ya[:, 0:16], wd0, preferred_element_type=jnp.float32)
    v1x = jnp.dot(ya[:, 16:32], wd1, preferred_element_type=jnp.float32)
    v1y = jnp.dot(yb[:, 0:16], wd1, preferred_element_type=jnp.float32)
    v1z = jnp.dot(yb[:, 16:32], wd1, preferred_element_type=jnp.float32)
    n2 = v1x * v1x + v1y * v1y + v1z * v1z
    wz = wsc_ref[0]
    s2 = s * s
    z0 = (wz[0:1, :] * s + wz[1:2, :] * s2 + wz[2:3, :] * (s2 * s)
          + wz[3:4, :] * n2 + wz[4:5, :] * (s * n2))
    t1 = wz[5:6, :] + wz[6:7, :] * s + wz[7:8, :] * s2 + wz[8:9, :] * n2
    z1x = t1 * v1x
    z1y = t1 * v1y
    z1z = t1 * v1z
    wp0 = wp0_ref[...]
    wp1 = wp1_ref[...]
    z0 = jnp.dot(z0, wp0, preferred_element_type=jnp.float32)
    z1x = jnp.dot(z1x, wp1, preferred_element_type=jnp.float32)
    z1y = jnp.dot(z1y, wp1, preferred_element_type=jnp.float32)
    z1z = jnp.dot(z1z, wp1, preferred_element_type=jnp.float32)
    nf1 = nf1_ref[...]
    wsk0 = wsk0_ref[0]
    wsk1 = wsk1_ref[0]
    z0 = z0 + jnp.dot(nf0_ref[...], wsk0, preferred_element_type=jnp.float32)
    z1x = z1x + jnp.dot(nf1[:, 0:16], wsk1, preferred_element_type=jnp.float32)
    z1y = z1y + jnp.dot(nf1[:, 16:32], wsk1,
                        preferred_element_type=jnp.float32)
    z1z = z1z + jnp.dot(nf1[:, 32:48], wsk1,
                        preferred_element_type=jnp.float32)
    z0_ref[...] = z0
    z1_ref[...] = jnp.concatenate([z1x, z1y, z1z], axis=1)
    ro_ref[...] = jnp.sum(z0 * wro_ref[...][:, 0][None, :], axis=1,
                          keepdims=True)


def _node_call(ya, yb, nf0, nf1p,
               wd0e, wd1e, wsk0, wsk1, wscp, wp0, wp1, wro):
    n = nf0.shape[0]
    nb = min(1000, n // S)
    bps = (n // S) // nb
    grid = n // nb

    return pl.pallas_call(
        _node_body,
        grid=(grid,),
        in_specs=[
            pl.BlockSpec((nb, 32), lambda i: (i, 0)),
            pl.BlockSpec((nb, 32), lambda i: (i, 0)),
            pl.BlockSpec((nb, F), lambda i: (i, 0)),
            pl.BlockSpec((nb, 3 * F), lambda i: (i, 0)),
            pl.BlockSpec((F, F), lambda i: (0, 0)),
            pl.BlockSpec((F, F), lambda i: (0, 0)),
            pl.BlockSpec((1, F, F), lambda i: (i // bps, 0, 0)),
            pl.BlockSpec((1, F, F), lambda i: (i // bps, 0, 0)),
            pl.BlockSpec((1, F, F), lambda i: (i // bps, 0, 0)),
            pl.BlockSpec((F, F), lambda i: (0, 0)),
            pl.BlockSpec((F, F), lambda i: (0, 0)),
            pl.BlockSpec((F, 1), lambda i: (0, 0)),
        ],
        out_specs=[
            pl.BlockSpec((nb, F), lambda i: (i, 0)),
            pl.BlockSpec((nb, 3 * F), lambda i: (i, 0)),
            pl.BlockSpec((nb, 1), lambda i: (i, 0)),
        ],
        out_shape=[
            jax.ShapeDtypeStruct((n, F), jnp.float32),
            jax.ShapeDtypeStruct((n, 3 * F), jnp.float32),
            jax.ShapeDtypeStruct((n, 1), jnp.float32),
        ],
    )(ya, yb, nf0, nf1p,
      wd0e, wd1e, wsk0, wsk1, wscp, wp0, wp1, wro)


def kernel(vectors, node_feats_l0, node_feats_l1, num_species_counts,
           radial_embeddings, senders, receivers, num_nodes,
           W_up_l0, W_up_l1, mlp_w0, mlp_w1, mlp_w2, mlp_w3,
           W_down_l0, W_down_l1, W_skip_l0, W_skip_l1, W_sc,
           W_post_l0, W_post_l1, W_ro):
    n = node_feats_l0.shape[0]
    e = vectors.shape[0]
    epb = _IDXW * _NS * _CHUNK_ROWS
    e_pad = ((e + epb - 1) // epb) * epb

    nf0 = node_feats_l0[:, :, 0]
    nf1p = jnp.transpose(node_feats_l1, (0, 2, 1)).reshape(n, 3 * F)

    h_tab = _up_call(nf0, nf1p, W_up_l0, W_up_l1)

    h1 = ((e_pad // epb + 1) // 2) * epb
    vecT = jnp.pad(vectors.T, ((0, 0), (0, e_pad - e)))

    zpad_i = jnp.zeros((e_pad - e,), jnp.int32)
    snd2 = jnp.concatenate([senders.astype(jnp.int32), zpad_i]).reshape(
        e_pad // _IDXW, _IDXW)
    rcv2 = jnp.concatenate([receivers.astype(jnp.int32), zpad_i]).reshape(
        e_pad // _IDXW, _IDXW)
    r1 = h1 // _IDXW

    p1 = _edge_call(radial_embeddings[:h1], h1,
                    mlp_w0, mlp_w1, mlp_w2, mlp_w3)
    s1 = _sh_call(vecT[:, :h1]).T
    ya1, yb1 = _sc_call(h_tab, p1, s1, snd2[:r1], rcv2[:r1])

    p2 = _edge_call(radial_embeddings[h1:], e_pad - h1,
                    mlp_w0, mlp_w1, mlp_w2, mlp_w3)
    s2 = _sh_call(vecT[:, h1:]).T
    ya, yb = _sc_call(h_tab, p2, s2, snd2[r1:], rcv2[r1:], ya1, yb1)

    wscp = jnp.concatenate(
        [W_sc, jnp.zeros((S, 7, F), jnp.float32)], axis=1)
    z0f, z1p, ro = _node_call(
        ya, yb, nf0, nf1p, W_down_l0 * EPS, W_down_l1 * EPS,
        W_skip_l0, W_skip_l1, wscp, W_post_l0, W_post_l1, W_ro)

    z0 = z0f[:, :, None]
    z1 = jnp.transpose(z1p.reshape(n, 3, F), (0, 2, 1))
    return z0, z1, ro


# speedup vs baseline: 1.4486x; 1.0681x over previous
import functools
import numpy as np
import jax
import jax.numpy as jnp
from jax import lax
from jax.experimental import pallas as pl
from jax.experimental.pallas import tpu as pltpu
from jax.experimental.pallas import tpu_sc as plsc

F = 16
S = 10
EPS = 0.25
_SQ3 = float(np.sqrt(3.0))

_NC = 2
_NS = 16
_IDXW = 128
_CHUNK_ROWS = 8
_CE = _IDXW
_NPAD = 50048


def _up_body(nf0_ref, nf1_ref, w0_ref, w1_ref, out_ref):
    w0 = w0_ref[...]
    w1 = w1_ref[...]
    h0 = jnp.dot(nf0_ref[...], w0, preferred_element_type=jnp.float32)
    nf1 = nf1_ref[...]
    h1x = jnp.dot(nf1[:, 0:16], w1, preferred_element_type=jnp.float32)
    h1y = jnp.dot(nf1[:, 16:32], w1, preferred_element_type=jnp.float32)
    h1z = jnp.dot(nf1[:, 32:48], w1, preferred_element_type=jnp.float32)
    out_ref[...] = jnp.concatenate([h0, h1x, h1y, h1z], axis=1)


def _up_call(nf0, nf1p, W_up_l0, W_up_l1):
    n = nf0.shape[0]
    nb = 2000
    grid = n // nb
    return pl.pallas_call(
        _up_body,
        grid=(grid,),
        in_specs=[
            pl.BlockSpec((nb, F), lambda i: (i, 0)),
            pl.BlockSpec((nb, 3 * F), lambda i: (i, 0)),
            pl.BlockSpec((F, F), lambda i: (0, 0)),
            pl.BlockSpec((F, F), lambda i: (0, 0)),
        ],
        out_specs=pl.BlockSpec((nb, 4 * F), lambda i: (i, 0)),
        out_shape=jax.ShapeDtypeStruct((n, 4 * F), jnp.float32),
    )(nf0, nf1p, W_up_l0, W_up_l1)


def _silu(x):
    return x * jax.nn.sigmoid(x)


def _edge_body(nvalid, r_ref, m0_ref, m1_ref, m2_ref, m3_ref, out_ref):
    r = r_ref[...]
    h = _silu(jnp.dot(r, m0_ref[...], preferred_element_type=jnp.float32))
    h = _silu(jnp.dot(h, m1_ref[...], preferred_element_type=jnp.float32))
    h = _silu(jnp.dot(h, m2_ref[...], preferred_element_type=jnp.float32))
    mix = jnp.dot(h, m3_ref[...], preferred_element_type=jnp.float32)
    wscale = jnp.concatenate([
        jnp.ones((16,), jnp.float32),
        jnp.full((16,), 1.0 / _SQ3, jnp.float32),
        jnp.ones((48,), jnp.float32),
    ])[None, :]
    out_ref[...] = mix * wscale
    eb = r_ref.shape[0]
    base = pl.program_id(0) * eb

    @pl.when(base + eb > nvalid)
    def _():
        rid = base + jax.lax.broadcasted_iota(jnp.int32, (eb, 80), 0)
        out_ref[...] = jnp.where(rid < nvalid, out_ref[...], 0.0)


def _edge_call(r8, epad, mlp_w0, mlp_w1, mlp_w2, mlp_w3):
    e = r8.shape[0]
    eb = 4096
    grid = epad // eb
    last_in = (e - 1) // eb
    return pl.pallas_call(
        functools.partial(_edge_body, e),
        grid=(grid,),
        in_specs=[
            pl.BlockSpec((eb, 8), lambda i: (jnp.minimum(i, last_in), 0)),
            pl.BlockSpec(mlp_w0.shape, lambda i: (0, 0)),
            pl.BlockSpec(mlp_w1.shape, lambda i: (0, 0)),
            pl.BlockSpec(mlp_w2.shape, lambda i: (0, 0)),
            pl.BlockSpec(mlp_w3.shape, lambda i: (0, 0)),
        ],
        out_specs=pl.BlockSpec((eb, 80), lambda i: (i, 0)),
        out_shape=jax.ShapeDtypeStruct((epad, 80), jnp.float32),
    )(r8, mlp_w0, mlp_w1, mlp_w2, mlp_w3)


def _sh_body(v_ref, out_ref):
    x = v_ref[0:1, :]
    y = v_ref[1:2, :]
    z = v_ref[2:3, :]
    rn = jnp.sqrt(x * x + y * y + z * z)
    inv = 1.0 / (rn + 1e-9)
    ux = x * inv
    uy = y * inv
    uz = z * inv
    shx = _SQ3 * ux
    shy = _SQ3 * uy
    shz = _SQ3 * uz
    a = _SQ3 * ux * uy
    b = _SQ3 * uy * uz
    c = 1.5 * uz * uz - 0.5
    d = _SQ3 * ux * uz
    e = 0.5 * _SQ3 * (ux * ux - uy * uy)
    m00 = e - 0.5 * c
    m11 = -e - 0.5 * c
    zpad = jnp.zeros((7, x.shape[1]), jnp.float32)
    out_ref[...] = jnp.concatenate(
        [shx, shy, shz, m00, a, d, m11, b, c, zpad], axis=0)


def _sh_call(vecT):
    epad = vecT.shape[1]
    ebT = 16384
    grid = epad // ebT
    return pl.pallas_call(
        _sh_body,
        grid=(grid,),
        in_specs=[pl.BlockSpec((3, ebT), lambda i: (0, i))],
        out_specs=pl.BlockSpec((16, ebT), lambda i: (0, i)),
        out_shape=jax.ShapeDtypeStruct((16, epad), jnp.float32),
    )(vecT)


def _sc_body(chained, h_hbm, p_hbm, s_hbm, snd_hbm, rcv_hbm, *rest):
    if chained:
        (yain_hbm, ybin_hbm, ya_hbm, yb_hbm,
         acc_sh, idx_s, idx_r, p_v, s_v, x_v, m_v, z_v,
         sem_p, sem_s, sem_g) = rest
    else:
        (ya_hbm, yb_hbm,
         acc_sh, idx_s, idx_r, p_v, s_v, x_v, m_v, z_v,
         sem_p, sem_s, sem_g) = rest
    core = lax.axis_index("c")
    sub = lax.axis_index("s")
    n = ya_hbm.shape[0]
    rows_per_sub_out = n // _NS
    zrows = z_v.shape[0]
    obase = sub * rows_per_sub_out

    if chained:
        @pl.when(core == 0)
        def _():
            pltpu.sync_copy(yain_hbm.at[pl.ds(obase, rows_per_sub_out)],
                            acc_sh.at[pl.ds(obase, rows_per_sub_out)])

        @pl.when(core == 1)
        def _():
            pltpu.sync_copy(ybin_hbm.at[pl.ds(obase, rows_per_sub_out)],
                            acc_sh.at[pl.ds(obase, rows_per_sub_out)])
    else:
        zero16 = jnp.zeros((16,), jnp.float32)

        def zinit(i, carry):
            z_v[i, 0:16] = zero16
            z_v[i, 16:32] = zero16
            return carry

        lax.fori_loop(0, zrows, zinit, 0)

        def zcopy(i, carry):
            pltpu.sync_copy(z_v, acc_sh.at[pl.ds(obase + i * zrows, zrows)])
            return carry

        lax.fori_loop(0, rows_per_sub_out // zrows, zcopy, 0)
    plsc.subcore_barrier()

    total_rows = snd_hbm.shape[0]
    rows_per_sub = total_rows // _NS
    row0 = sub * rows_per_sub

    def compute_core0(e4, carry):
        for j in range(1):
            e = e4 + j
            x0 = x_v[e, 0:16]
            xx = x_v[e, 16:32]
            xy = x_v[e, 32:48]
            xz = x_v[e, 48:64]
            w0 = p_v[e, 0:16]
            w1 = p_v[e, 16:32]
            w2 = p_v[e, 32:48]
            w3 = p_v[e, 48:64]
            w4 = p_v[e, 64:80]
            sv = s_v[e, 0:16]
            shx = sv[0]
            shy = sv[1]
            shz = sv[2]
            m00 = sv[3]
            m01 = sv[4]
            m02 = sv[5]
            dot = xx * shx + xy * shy + xz * shz
            m0 = w0 * x0 + w1 * dot
            t = w2 * x0
            m1x = t * shx + w3 * xx + w4 * (xx * m00 + xy * m01 + xz * m02)
            m_v[e, 0:16] = m0
            m_v[e, 16:32] = m1x
        return carry

    def compute_core1(e4, carry):
        for j in range(1):
            e = e4 + j
            x0 = x_v[e, 0:16]
            xx = x_v[e, 16:32]
            xy = x_v[e, 32:48]
            xz = x_v[e, 48:64]
            w2 = p_v[e, 32:48]
            w3 = p_v[e, 48:64]
            w4 = p_v[e, 64:80]
            sv = s_v[e, 0:16]
            shy = sv[1]
            shz = sv[2]
            m01 = sv[4]
            m02 = sv[5]
            m11 = sv[6]
            m12 = sv[7]
            m22 = sv[8]
            t = w2 * x0
            m1y = t * shy + w3 * xy + w4 * (xx * m01 + xy * m11 + xz * m12)
            m1z = t * shz + w3 * xz + w4 * (xx * m02 + xy * m12 + xz * m22)
            m_v[e, 0:16] = m1y
            m_v[e, 16:32] = m1z
        return carry

    def chunk(k, carry):
        rbase = row0 + k * _CHUNK_ROWS
        pltpu.sync_copy(snd_hbm.at[pl.ds(rbase, _CHUNK_ROWS)], idx_s)
        pltpu.sync_copy(rcv_hbm.at[pl.ds(rbase, _CHUNK_ROWS)], idx_r)
        for h in range(_CHUNK_ROWS):
            cp = pltpu.async_copy(
                p_hbm.at[pl.ds((rbase + h) * _IDXW, _CE)], p_v, sem_p)
            cs = pltpu.async_copy(
                s_hbm.at[pl.ds((rbase + h) * _IDXW, _CE)], s_v, sem_s)
            gd = pltpu.async_copy(h_hbm.at[idx_s.at[h]], x_v, sem_g)
            cp.wait()
            cs.wait()
            gd.wait()

            @pl.when(core == 0)
            def _():
                lax.fori_loop(0, _CE, compute_core0, 0)

            @pl.when(core == 1)
            def _():
                lax.fori_loop(0, _CE, compute_core1, 0)

            pltpu.sync_copy(m_v, acc_sh.at[idx_r.at[h]], add=True)
        return carry

    lax.fori_loop(0, rows_per_sub // _CHUNK_ROWS, chunk, 0)
    plsc.subcore_barrier()

    @pl.when(core == 0)
    def _():
        pltpu.sync_copy(acc_sh.at[pl.ds(obase, rows_per_sub_out)],
                        ya_hbm.at[pl.ds(obase, rows_per_sub_out)])

    @pl.when(core == 1)
    def _():
        pltpu.sync_copy(acc_sh.at[pl.ds(obase, rows_per_sub_out)],
                        yb_hbm.at[pl.ds(obase, rows_per_sub_out)])


def _sc_call(h_tab, p_edge, s_edge, snd2, rcv2, ya_in=None, yb_in=None):
    mesh = plsc.VectorSubcoreMesh(
        core_axis_name="c", subcore_axis_name="s",
        num_cores=_NC, num_subcores=_NS)
    chained = ya_in is not None
    fn = functools.partial(
        pl.kernel,
        out_type=[
            jax.ShapeDtypeStruct((_NPAD, 32), jnp.float32),
            jax.ShapeDtypeStruct((_NPAD, 32), jnp.float32),
        ],
        mesh=mesh,
        scratch_types=[
            pltpu.VMEM_SHARED((_NPAD, 32), jnp.float32),
            pltpu.VMEM((_CHUNK_ROWS, _IDXW), jnp.int32),
            pltpu.VMEM((_CHUNK_ROWS, _IDXW), jnp.int32),
            pltpu.VMEM((_CE, 80), jnp.float32),
            pltpu.VMEM((_CE, 16), jnp.float32),
            pltpu.VMEM((_CE, 64), jnp.float32),
            pltpu.VMEM((_CE, 32), jnp.float32),
            pltpu.VMEM((8, 32), jnp.float32),
            pltpu.SemaphoreType.DMA,
            pltpu.SemaphoreType.DMA,
            pltpu.SemaphoreType.DMA,
        ],
        compiler_params=pltpu.CompilerParams(use_tc_tiling_on_sc=False),
    )(functools.partial(_sc_body, chained))
    if chained:
        return fn(h_tab, p_edge, s_edge, snd2, rcv2, ya_in, yb_in)
    return fn(h_tab, p_edge, s_edge, snd2, rcv2)


def _node_body(ya_ref, yb_ref, nf0_ref, nf1_ref,
               wd0_ref, wd1_ref, wsk0_ref, wsk1_ref, wsc_ref,
               wp0_ref, wp1_ref, wro_ref,
               z0_ref, z1_ref, ro_ref):
    ya = ya_ref[...]
    yb = yb_ref[...]
    wd0 = wd0_ref[...]
    wd1 = wd1_ref[...]
    s = jnp.dot(ya[:, 0:16], wd0, preferred_element_type=jnp.float32)
    v1x = jnp.dot(ya[:, 16:32], wd1, preferred_element_type=jnp.float32)
    v1y = jnp.dot(yb[:, 0:16], wd1, preferred_element_type=jnp.float32)
    v1z = jnp.dot(yb[:, 16:32], wd1, preferred_element_type=jnp.float32)
    n2 = v1x * v1x + v1y * v1y + v1z * v1z
    wz = wsc_ref[0]
    s2 = s * s
    z0 = (wz[0:1, :] * s + wz[1:2, :] * s2 + wz[2:3, :] * (s2 * s)
          + wz[3:4, :] * n2 + wz[4:5, :] * (s * n2))
    t1 = wz[5:6, :] + wz[6:7, :] * s + wz[7:8, :] * s2 + wz[8:9, :] * n2
    z1x = t1 * v1x
    z1y = t1 * v1y
    z1z = t1 * v1z
    wp0 = wp0_ref[...]
    wp1 = wp1_ref[...]
    z0 = jnp.dot(z0, wp0, preferred_element_type=jnp.float32)
    z1x = jnp.dot(z1x, wp1, preferred_element_type=jnp.float32)
    z1y = jnp.dot(z1y, wp1, preferred_element_type=jnp.float32)
    z1z = jnp.dot(z1z, wp1, preferred_element_type=jnp.float32)
    nf1 = nf1_ref[...]
    wsk0 = wsk0_ref[0]
    wsk1 = wsk1_ref[0]
    z0 = z0 + jnp.dot(nf0_ref[...], wsk0, preferred_element_type=jnp.float32)
    z1x = z1x + jnp.dot(nf1[:, 0:16], wsk1, preferred_element_type=jnp.float32)
    z1y = z1y + jnp.dot(nf1[:, 16:32], wsk1,
                        preferred_element_type=jnp.float32)
    z1z = z1z + jnp.dot(nf1[:, 32:48], wsk1,
                        preferred_element_type=jnp.float32)
    z0_ref[...] = z0
    z1_ref[...] = jnp.concatenate([z1x, z1y, z1z], axis=1)
    ro_ref[...] = jnp.sum(z0 * wro_ref[...][:, 0][None, :], axis=1,
                          keepdims=True)


def _node_call(ya, yb, nf0, nf1p,
               wd0e, wd1e, wsk0, wsk1, wscp, wp0, wp1, wro):
    n = nf0.shape[0]
    nb = min(1000, n // S)
    bps = (n // S) // nb
    grid = n // nb

    return pl.pallas_call(
        _node_body,
        grid=(grid,),
        in_specs=[
            pl.BlockSpec((nb, 32), lambda i: (i, 0)),
            pl.BlockSpec((nb, 32), lambda i: (i, 0)),
            pl.BlockSpec((nb, F), lambda i: (i, 0)),
            pl.BlockSpec((nb, 3 * F), lambda i: (i, 0)),
            pl.BlockSpec((F, F), lambda i: (0, 0)),
            pl.BlockSpec((F, F), lambda i: (0, 0)),
            pl.BlockSpec((1, F, F), lambda i: (i // bps, 0, 0)),
            pl.BlockSpec((1, F, F), lambda i: (i // bps, 0, 0)),
            pl.BlockSpec((1, F, F), lambda i: (i // bps, 0, 0)),
            pl.BlockSpec((F, F), lambda i: (0, 0)),
            pl.BlockSpec((F, F), lambda i: (0, 0)),
            pl.BlockSpec((F, 1), lambda i: (0, 0)),
        ],
        out_specs=[
            pl.BlockSpec((nb, F), lambda i: (i, 0)),
            pl.BlockSpec((nb, 3 * F), lambda i: (i, 0)),
            pl.BlockSpec((nb, 1), lambda i: (i, 0)),
        ],
        out_shape=[
            jax.ShapeDtypeStruct((n, F), jnp.float32),
            jax.ShapeDtypeStruct((n, 3 * F), jnp.float32),
            jax.ShapeDtypeStruct((n, 1), jnp.float32),
        ],
    )(ya, yb, nf0, nf1p,
      wd0e, wd1e, wsk0, wsk1, wscp, wp0, wp1, wro)


def kernel(vectors, node_feats_l0, node_feats_l1, num_species_counts,
           radial_embeddings, senders, receivers, num_nodes,
           W_up_l0, W_up_l1, mlp_w0, mlp_w1, mlp_w2, mlp_w3,
           W_down_l0, W_down_l1, W_skip_l0, W_skip_l1, W_sc,
           W_post_l0, W_post_l1, W_ro):
    n = node_feats_l0.shape[0]
    e = vectors.shape[0]
    epb = _IDXW * _NS * _CHUNK_ROWS
    e_pad = ((e + epb - 1) // epb) * epb

    nf0 = node_feats_l0[:, :, 0]
    nf1p = jnp.transpose(node_feats_l1, (0, 2, 1)).reshape(n, 3 * F)

    h_tab = _up_call(nf0, nf1p, W_up_l0, W_up_l1)

    vecT = jnp.pad(vectors.T, ((0, 0), (0, e_pad - e)))

    zpad_i = jnp.zeros((e_pad - e,), jnp.int32)
    snd2 = jnp.concatenate([senders.astype(jnp.int32), zpad_i]).reshape(
        e_pad // _IDXW, _IDXW)
    rcv2 = jnp.concatenate([receivers.astype(jnp.int32), zpad_i]).reshape(
        e_pad // _IDXW, _IDXW)

    nch = 3
    gr = e_pad // epb
    sizes = [(gr // nch + (1 if i < gr % nch else 0)) * epb
             for i in range(nch)]
    ya = yb = None
    off = 0
    for sz in sizes:
        p_k = _edge_call(radial_embeddings[off:off + sz], sz,
                         mlp_w0, mlp_w1, mlp_w2, mlp_w3)
        s_k = _sh_call(vecT[:, off:off + sz]).T
        r0 = off // _IDXW
        r1 = (off + sz) // _IDXW
        ya, yb = _sc_call(h_tab, p_k, s_k, snd2[r0:r1], rcv2[r0:r1], ya, yb)
        off += sz

    wscp = jnp.concatenate(
        [W_sc, jnp.zeros((S, 7, F), jnp.float32)], axis=1)
    z0f, z1p, ro = _node_call(
        ya, yb, nf0, nf1p, W_down_l0 * EPS, W_down_l1 * EPS,
        W_skip_l0, W_skip_l1, wscp, W_post_l0, W_post_l1, W_ro)

    z0 = z0f[:, :, None]
    z1 = jnp.transpose(z1p.reshape(n, 3, F), (0, 2, 1))
    return z0, z1, ro


# speedup vs baseline: 1.5030x; 1.0375x over previous
import functools
import numpy as np
import jax
import jax.numpy as jnp
from jax import lax
from jax.experimental import pallas as pl
from jax.experimental.pallas import tpu as pltpu
from jax.experimental.pallas import tpu_sc as plsc

F = 16
S = 10
EPS = 0.25
_SQ3 = float(np.sqrt(3.0))

_NC = 2
_NS = 16
_IDXW = 128
_CHUNK_ROWS = 8
_CE = _IDXW
_NPAD = 50048


def _up_body(nf0_ref, nf1_ref, w0_ref, w1_ref, out_ref):
    w0 = w0_ref[...]
    w1 = w1_ref[...]
    h0 = jnp.dot(nf0_ref[...], w0, preferred_element_type=jnp.float32)
    nf1 = nf1_ref[...]
    h1x = jnp.dot(nf1[:, 0:16], w1, preferred_element_type=jnp.float32)
    h1y = jnp.dot(nf1[:, 16:32], w1, preferred_element_type=jnp.float32)
    h1z = jnp.dot(nf1[:, 32:48], w1, preferred_element_type=jnp.float32)
    out_ref[...] = jnp.concatenate([h0, h1x, h1y, h1z], axis=1)


def _up_call(nf0, nf1p, W_up_l0, W_up_l1):
    n = nf0.shape[0]
    nb = 2000
    grid = n // nb
    return pl.pallas_call(
        _up_body,
        grid=(grid,),
        in_specs=[
            pl.BlockSpec((nb, F), lambda i: (i, 0)),
            pl.BlockSpec((nb, 3 * F), lambda i: (i, 0)),
            pl.BlockSpec((F, F), lambda i: (0, 0)),
            pl.BlockSpec((F, F), lambda i: (0, 0)),
        ],
        out_specs=pl.BlockSpec((nb, 4 * F), lambda i: (i, 0)),
        out_shape=jax.ShapeDtypeStruct((n, 4 * F), jnp.float32),
    )(nf0, nf1p, W_up_l0, W_up_l1)


def _silu(x):
    return x * jax.nn.sigmoid(x)


def _edge_body(nvalid, r_ref, m0_ref, m1_ref, m2_ref, m3_ref, out_ref):
    r = r_ref[...]
    h = _silu(jnp.dot(r, m0_ref[...], preferred_element_type=jnp.float32))
    h = _silu(jnp.dot(h, m1_ref[...], preferred_element_type=jnp.float32))
    h = _silu(jnp.dot(h, m2_ref[...], preferred_element_type=jnp.float32))
    mix = jnp.dot(h, m3_ref[...], preferred_element_type=jnp.float32)
    wscale = jnp.concatenate([
        jnp.ones((16,), jnp.float32),
        jnp.full((16,), 1.0 / _SQ3, jnp.float32),
        jnp.ones((48,), jnp.float32),
    ])[None, :]
    out_ref[...] = mix * wscale
    eb = r_ref.shape[0]
    base = pl.program_id(0) * eb

    @pl.when(base + eb > nvalid)
    def _():
        rid = base + jax.lax.broadcasted_iota(jnp.int32, (eb, 80), 0)
        out_ref[...] = jnp.where(rid < nvalid, out_ref[...], 0.0)


def _edge_call(r8, epad, mlp_w0, mlp_w1, mlp_w2, mlp_w3):
    e = r8.shape[0]
    eb = 4096
    grid = epad // eb
    last_in = (e - 1) // eb
    return pl.pallas_call(
        functools.partial(_edge_body, e),
        grid=(grid,),
        in_specs=[
            pl.BlockSpec((eb, 8), lambda i: (jnp.minimum(i, last_in), 0)),
            pl.BlockSpec(mlp_w0.shape, lambda i: (0, 0)),
            pl.BlockSpec(mlp_w1.shape, lambda i: (0, 0)),
            pl.BlockSpec(mlp_w2.shape, lambda i: (0, 0)),
            pl.BlockSpec(mlp_w3.shape, lambda i: (0, 0)),
        ],
        out_specs=pl.BlockSpec((eb, 80), lambda i: (i, 0)),
        out_shape=jax.ShapeDtypeStruct((epad, 80), jnp.float32),
    )(r8, mlp_w0, mlp_w1, mlp_w2, mlp_w3)


def _sh_body(v_ref, out_ref):
    x = v_ref[0:1, :]
    y = v_ref[1:2, :]
    z = v_ref[2:3, :]
    rn = jnp.sqrt(x * x + y * y + z * z)
    inv = 1.0 / (rn + 1e-9)
    ux = x * inv
    uy = y * inv
    uz = z * inv
    shx = _SQ3 * ux
    shy = _SQ3 * uy
    shz = _SQ3 * uz
    a = _SQ3 * ux * uy
    b = _SQ3 * uy * uz
    c = 1.5 * uz * uz - 0.5
    d = _SQ3 * ux * uz
    e = 0.5 * _SQ3 * (ux * ux - uy * uy)
    m00 = e - 0.5 * c
    m11 = -e - 0.5 * c
    zpad = jnp.zeros((7, x.shape[1]), jnp.float32)
    out_ref[...] = jnp.concatenate(
        [shx, shy, shz, m00, a, d, m11, b, c, zpad], axis=0)


def _sh_call(vecT):
    epad = vecT.shape[1]
    ebT = 16384
    grid = epad // ebT
    return pl.pallas_call(
        _sh_body,
        grid=(grid,),
        in_specs=[pl.BlockSpec((3, ebT), lambda i: (0, i))],
        out_specs=pl.BlockSpec((16, ebT), lambda i: (0, i)),
        out_shape=jax.ShapeDtypeStruct((16, epad), jnp.float32),
    )(vecT)


def _sc_body(chained, h_hbm, p_hbm, s_hbm, snd_hbm, rcv_hbm, *rest):
    if chained:
        (yain_hbm, ybin_hbm, ya_hbm, yb_hbm,
         acc_sh, idx_s, idx_r, p_v, s_v, x_v, m_v, z_v,
         sem_p, sem_s, sem_g) = rest
    else:
        (ya_hbm, yb_hbm,
         acc_sh, idx_s, idx_r, p_v, s_v, x_v, m_v, z_v,
         sem_p, sem_s, sem_g) = rest
    core = lax.axis_index("c")
    sub = lax.axis_index("s")
    n = ya_hbm.shape[0]
    rows_per_sub_out = n // _NS
    zrows = z_v.shape[0]
    obase = sub * rows_per_sub_out

    if chained:
        @pl.when(core == 0)
        def _():
            pltpu.sync_copy(yain_hbm.at[pl.ds(obase, rows_per_sub_out)],
                            acc_sh.at[pl.ds(obase, rows_per_sub_out)])

        @pl.when(core == 1)
        def _():
            pltpu.sync_copy(ybin_hbm.at[pl.ds(obase, rows_per_sub_out)],
                            acc_sh.at[pl.ds(obase, rows_per_sub_out)])
    else:
        zero16 = jnp.zeros((16,), jnp.float32)

        def zinit(i, carry):
            z_v[i, 0:16] = zero16
            z_v[i, 16:32] = zero16
            return carry

        lax.fori_loop(0, zrows, zinit, 0)

        def zcopy(i, carry):
            pltpu.sync_copy(z_v, acc_sh.at[pl.ds(obase + i * zrows, zrows)])
            return carry

        lax.fori_loop(0, rows_per_sub_out // zrows, zcopy, 0)
    plsc.subcore_barrier()

    total_rows = snd_hbm.shape[0]
    rows_per_sub = total_rows // _NS
    row0 = sub * rows_per_sub

    def compute_core0(e4, carry):
        for j in range(1):
            e = e4 + j
            x0 = x_v[e, 0:16]
            xx = x_v[e, 16:32]
            xy = x_v[e, 32:48]
            xz = x_v[e, 48:64]
            w0 = p_v[e, 0:16]
            w1 = p_v[e, 16:32]
            w2 = p_v[e, 32:48]
            w3 = p_v[e, 48:64]
            w4 = p_v[e, 64:80]
            sv = s_v[e, 0:16]
            shx = sv[0]
            shy = sv[1]
            shz = sv[2]
            m00 = sv[3]
            m01 = sv[4]
            m02 = sv[5]
            dot = xx * shx + xy * shy + xz * shz
            m0 = w0 * x0 + w1 * dot
            t = w2 * x0
            m1x = t * shx + w3 * xx + w4 * (xx * m00 + xy * m01 + xz * m02)
            m_v[e, 0:16] = m0
            m_v[e, 16:32] = m1x
        return carry

    def compute_core1(e4, carry):
        for j in range(1):
            e = e4 + j
            x0 = x_v[e, 0:16]
            xx = x_v[e, 16:32]
            xy = x_v[e, 32:48]
            xz = x_v[e, 48:64]
            w2 = p_v[e, 32:48]
            w3 = p_v[e, 48:64]
            w4 = p_v[e, 64:80]
            sv = s_v[e, 0:16]
            shy = sv[1]
            shz = sv[2]
            m01 = sv[4]
            m02 = sv[5]
            m11 = sv[6]
            m12 = sv[7]
            m22 = sv[8]
            t = w2 * x0
            m1y = t * shy + w3 * xy + w4 * (xx * m01 + xy * m11 + xz * m12)
            m1z = t * shz + w3 * xz + w4 * (xx * m02 + xy * m12 + xz * m22)
            m_v[e, 0:16] = m1y
            m_v[e, 16:32] = m1z
        return carry

    def chunk(k, carry):
        rbase = row0 + k * _CHUNK_ROWS
        pltpu.sync_copy(snd_hbm.at[pl.ds(rbase, _CHUNK_ROWS)], idx_s)
        pltpu.sync_copy(rcv_hbm.at[pl.ds(rbase, _CHUNK_ROWS)], idx_r)
        for h in range(_CHUNK_ROWS):
            cp = pltpu.async_copy(
                p_hbm.at[pl.ds((rbase + h) * _IDXW, _CE)], p_v, sem_p)
            cs = pltpu.async_copy(
                s_hbm.at[pl.ds((rbase + h) * _IDXW, _CE)], s_v, sem_s)
            gd = pltpu.async_copy(h_hbm.at[idx_s.at[h]], x_v, sem_g)
            cp.wait()
            cs.wait()
            gd.wait()

            @pl.when(core == 0)
            def _():
                lax.fori_loop(0, _CE, compute_core0, 0)

            @pl.when(core == 1)
            def _():
                lax.fori_loop(0, _CE, compute_core1, 0)

            pltpu.sync_copy(m_v, acc_sh.at[idx_r.at[h]], add=True)
        return carry

    lax.fori_loop(0, rows_per_sub // _CHUNK_ROWS, chunk, 0)
    plsc.subcore_barrier()

    @pl.when(core == 0)
    def _():
        pltpu.sync_copy(acc_sh.at[pl.ds(obase, rows_per_sub_out)],
                        ya_hbm.at[pl.ds(obase, rows_per_sub_out)])

    @pl.when(core == 1)
    def _():
        pltpu.sync_copy(acc_sh.at[pl.ds(obase, rows_per_sub_out)],
                        yb_hbm.at[pl.ds(obase, rows_per_sub_out)])


def _sc_call(h_tab, p_edge, s_edge, snd2, rcv2, ya_in=None, yb_in=None):
    mesh = plsc.VectorSubcoreMesh(
        core_axis_name="c", subcore_axis_name="s",
        num_cores=_NC, num_subcores=_NS)
    chained = ya_in is not None
    fn = functools.partial(
        pl.kernel,
        out_type=[
            jax.ShapeDtypeStruct((_NPAD, 32), jnp.float32),
            jax.ShapeDtypeStruct((_NPAD, 32), jnp.float32),
        ],
        mesh=mesh,
        scratch_types=[
            pltpu.VMEM_SHARED((_NPAD, 32), jnp.float32),
            pltpu.VMEM((_CHUNK_ROWS, _IDXW), jnp.int32),
            pltpu.VMEM((_CHUNK_ROWS, _IDXW), jnp.int32),
            pltpu.VMEM((_CE, 80), jnp.float32),
            pltpu.VMEM((_CE, 16), jnp.float32),
            pltpu.VMEM((_CE, 64), jnp.float32),
            pltpu.VMEM((_CE, 32), jnp.float32),
            pltpu.VMEM((8, 32), jnp.float32),
            pltpu.SemaphoreType.DMA,
            pltpu.SemaphoreType.DMA,
            pltpu.SemaphoreType.DMA,
        ],
        compiler_params=pltpu.CompilerParams(use_tc_tiling_on_sc=False),
    )(functools.partial(_sc_body, chained))
    if chained:
        return fn(h_tab, p_edge, s_edge, snd2, rcv2, ya_in, yb_in)
    return fn(h_tab, p_edge, s_edge, snd2, rcv2)


def _node_body(ya_ref, yb_ref, nf0_ref, nf1_ref,
               wd0_ref, wd1_ref, wsk0_ref, wsk1_ref, wsc_ref,
               wp0_ref, wp1_ref, wro_ref,
               z0_ref, z1_ref, ro_ref):
    ya = ya_ref[...]
    yb = yb_ref[...]
    wd0 = wd0_ref[...]
    wd1 = wd1_ref[...]
    s = jnp.dot(ya[:, 0:16], wd0, preferred_element_type=jnp.float32)
    v1x = jnp.dot(ya[:, 16:32], wd1, preferred_element_type=jnp.float32)
    v1y = jnp.dot(yb[:, 0:16], wd1, preferred_element_type=jnp.float32)
    v1z = jnp.dot(yb[:, 16:32], wd1, preferred_element_type=jnp.float32)
    n2 = v1x * v1x + v1y * v1y + v1z * v1z
    wz = wsc_ref[0]
    s2 = s * s
    z0 = (wz[0:1, :] * s + wz[1:2, :] * s2 + wz[2:3, :] * (s2 * s)
          + wz[3:4, :] * n2 + wz[4:5, :] * (s * n2))
    t1 = wz[5:6, :] + wz[6:7, :] * s + wz[7:8, :] * s2 + wz[8:9, :] * n2
    z1x = t1 * v1x
    z1y = t1 * v1y
    z1z = t1 * v1z
    wp0 = wp0_ref[...]
    wp1 = wp1_ref[...]
    z0 = jnp.dot(z0, wp0, preferred_element_type=jnp.float32)
    z1x = jnp.dot(z1x, wp1, preferred_element_type=jnp.float32)
    z1y = jnp.dot(z1y, wp1, preferred_element_type=jnp.float32)
    z1z = jnp.dot(z1z, wp1, preferred_element_type=jnp.float32)
    nf1 = nf1_ref[...]
    wsk0 = wsk0_ref[0]
    wsk1 = wsk1_ref[0]
    z0 = z0 + jnp.dot(nf0_ref[...], wsk0, preferred_element_type=jnp.float32)
    z1x = z1x + jnp.dot(nf1[:, 0:16], wsk1, preferred_element_type=jnp.float32)
    z1y = z1y + jnp.dot(nf1[:, 16:32], wsk1,
                        preferred_element_type=jnp.float32)
    z1z = z1z + jnp.dot(nf1[:, 32:48], wsk1,
                        preferred_element_type=jnp.float32)
    z0_ref[...] = z0
    z1_ref[...] = jnp.concatenate([z1x, z1y, z1z], axis=1)
    ro_ref[...] = jnp.sum(z0 * wro_ref[...][:, 0][None, :], axis=1,
                          keepdims=True)


def _node_call(ya, yb, nf0, nf1p,
               wd0e, wd1e, wsk0, wsk1, wscp, wp0, wp1, wro):
    n = nf0.shape[0]
    nb = min(1000, n // S)
    bps = (n // S) // nb
    grid = n // nb

    return pl.pallas_call(
        _node_body,
        grid=(grid,),
        in_specs=[
            pl.BlockSpec((nb, 32), lambda i: (i, 0)),
            pl.BlockSpec((nb, 32), lambda i: (i, 0)),
            pl.BlockSpec((nb, F), lambda i: (i, 0)),
            pl.BlockSpec((nb, 3 * F), lambda i: (i, 0)),
            pl.BlockSpec((F, F), lambda i: (0, 0)),
            pl.BlockSpec((F, F), lambda i: (0, 0)),
            pl.BlockSpec((1, F, F), lambda i: (i // bps, 0, 0)),
            pl.BlockSpec((1, F, F), lambda i: (i // bps, 0, 0)),
            pl.BlockSpec((1, F, F), lambda i: (i // bps, 0, 0)),
            pl.BlockSpec((F, F), lambda i: (0, 0)),
            pl.BlockSpec((F, F), lambda i: (0, 0)),
            pl.BlockSpec((F, 1), lambda i: (0, 0)),
        ],
        out_specs=[
            pl.BlockSpec((nb, F), lambda i: (i, 0)),
            pl.BlockSpec((nb, 3 * F), lambda i: (i, 0)),
            pl.BlockSpec((nb, 1), lambda i: (i, 0)),
        ],
        out_shape=[
            jax.ShapeDtypeStruct((n, F), jnp.float32),
            jax.ShapeDtypeStruct((n, 3 * F), jnp.float32),
            jax.ShapeDtypeStruct((n, 1), jnp.float32),
        ],
    )(ya, yb, nf0, nf1p,
      wd0e, wd1e, wsk0, wsk1, wscp, wp0, wp1, wro)


def kernel(vectors, node_feats_l0, node_feats_l1, num_species_counts,
           radial_embeddings, senders, receivers, num_nodes,
           W_up_l0, W_up_l1, mlp_w0, mlp_w1, mlp_w2, mlp_w3,
           W_down_l0, W_down_l1, W_skip_l0, W_skip_l1, W_sc,
           W_post_l0, W_post_l1, W_ro):
    n = node_feats_l0.shape[0]
    e = vectors.shape[0]
    epb = _IDXW * _NS * _CHUNK_ROWS
    e_pad = ((e + epb - 1) // epb) * epb

    nf0 = node_feats_l0[:, :, 0]
    nf1p = jnp.transpose(node_feats_l1, (0, 2, 1)).reshape(n, 3 * F)

    h_tab = _up_call(nf0, nf1p, W_up_l0, W_up_l1)

    vecT = jnp.pad(vectors.T, ((0, 0), (0, e_pad - e)))

    zpad_i = jnp.zeros((e_pad - e,), jnp.int32)
    snd2 = jnp.concatenate([senders.astype(jnp.int32), zpad_i]).reshape(
        e_pad // _IDXW, _IDXW)
    rcv2 = jnp.concatenate([receivers.astype(jnp.int32), zpad_i]).reshape(
        e_pad // _IDXW, _IDXW)

    nch = 4
    gr = e_pad // epb
    sizes = [(gr // nch + (1 if i < gr % nch else 0)) * epb
             for i in range(nch)]
    ya = yb = None
    off = 0
    for sz in sizes:
        p_k = _edge_call(radial_embeddings[off:off + sz], sz,
                         mlp_w0, mlp_w1, mlp_w2, mlp_w3)
        s_k = _sh_call(vecT[:, off:off + sz]).T
        r0 = off // _IDXW
        r1 = (off + sz) // _IDXW
        ya, yb = _sc_call(h_tab, p_k, s_k, snd2[r0:r1], rcv2[r0:r1], ya, yb)
        off += sz

    wscp = jnp.concatenate(
        [W_sc, jnp.zeros((S, 7, F), jnp.float32)], axis=1)
    z0f, z1p, ro = _node_call(
        ya, yb, nf0, nf1p, W_down_l0 * EPS, W_down_l1 * EPS,
        W_skip_l0, W_skip_l1, wscp, W_post_l0, W_post_l1, W_ro)

    z0 = z0f[:, :, None]
    z1 = jnp.transpose(z1p.reshape(n, 3, F), (0, 2, 1))
    return z0, z1, ro


# speedup vs baseline: 1.5372x; 1.0227x over previous
import functools
import numpy as np
import jax
import jax.numpy as jnp
from jax import lax
from jax.experimental import pallas as pl
from jax.experimental.pallas import tpu as pltpu
from jax.experimental.pallas import tpu_sc as plsc

F = 16
S = 10
EPS = 0.25
_SQ3 = float(np.sqrt(3.0))

_NC = 2
_NS = 16
_IDXW = 128
_CHUNK_ROWS = 8
_CE = _IDXW
_NPAD = 50048


def _up_body(nf0_ref, nf1_ref, w0_ref, w1_ref, out_ref):
    w0 = w0_ref[...]
    w1 = w1_ref[...]
    h0 = jnp.dot(nf0_ref[...], w0, preferred_element_type=jnp.float32)
    nf1 = nf1_ref[...]
    h1x = jnp.dot(nf1[:, 0:16], w1, preferred_element_type=jnp.float32)
    h1y = jnp.dot(nf1[:, 16:32], w1, preferred_element_type=jnp.float32)
    h1z = jnp.dot(nf1[:, 32:48], w1, preferred_element_type=jnp.float32)
    out_ref[...] = jnp.concatenate([h0, h1x, h1y, h1z], axis=1)


def _up_call(nf0, nf1p, W_up_l0, W_up_l1):
    n = nf0.shape[0]
    nb = 2000
    grid = n // nb
    return pl.pallas_call(
        _up_body,
        grid=(grid,),
        in_specs=[
            pl.BlockSpec((nb, F), lambda i: (i, 0)),
            pl.BlockSpec((nb, 3 * F), lambda i: (i, 0)),
            pl.BlockSpec((F, F), lambda i: (0, 0)),
            pl.BlockSpec((F, F), lambda i: (0, 0)),
        ],
        out_specs=pl.BlockSpec((nb, 4 * F), lambda i: (i, 0)),
        out_shape=jax.ShapeDtypeStruct((n, 4 * F), jnp.float32),
    )(nf0, nf1p, W_up_l0, W_up_l1)


def _silu(x):
    return x * jax.nn.sigmoid(x)


def _edge_body(nvalid, r_ref, m0_ref, m1_ref, m2_ref, m3_ref, out_ref):
    r = r_ref[...]
    h = _silu(jnp.dot(r, m0_ref[...], preferred_element_type=jnp.float32))
    h = _silu(jnp.dot(h, m1_ref[...], preferred_element_type=jnp.float32))
    h = _silu(jnp.dot(h, m2_ref[...], preferred_element_type=jnp.float32))
    mix = jnp.dot(h, m3_ref[...], preferred_element_type=jnp.float32)
    wscale = jnp.concatenate([
        jnp.ones((16,), jnp.float32),
        jnp.full((16,), 1.0 / _SQ3, jnp.float32),
        jnp.ones((48,), jnp.float32),
    ])[None, :]
    out_ref[...] = mix * wscale
    eb = r_ref.shape[0]
    base = pl.program_id(0) * eb

    @pl.when(base + eb > nvalid)
    def _():
        rid = base + jax.lax.broadcasted_iota(jnp.int32, (eb, 80), 0)
        out_ref[...] = jnp.where(rid < nvalid, out_ref[...], 0.0)


def _edge_call(r8, epad, mlp_w0, mlp_w1, mlp_w2, mlp_w3):
    e = r8.shape[0]
    eb = 4096
    grid = epad // eb
    last_in = (e - 1) // eb
    return pl.pallas_call(
        functools.partial(_edge_body, e),
        grid=(grid,),
        in_specs=[
            pl.BlockSpec((eb, 8), lambda i: (jnp.minimum(i, last_in), 0)),
            pl.BlockSpec(mlp_w0.shape, lambda i: (0, 0)),
            pl.BlockSpec(mlp_w1.shape, lambda i: (0, 0)),
            pl.BlockSpec(mlp_w2.shape, lambda i: (0, 0)),
            pl.BlockSpec(mlp_w3.shape, lambda i: (0, 0)),
        ],
        out_specs=pl.BlockSpec((eb, 80), lambda i: (i, 0)),
        out_shape=jax.ShapeDtypeStruct((epad, 80), jnp.float32),
    )(r8, mlp_w0, mlp_w1, mlp_w2, mlp_w3)


def _sh_body(v_ref, out_ref):
    x = v_ref[0:1, :]
    y = v_ref[1:2, :]
    z = v_ref[2:3, :]
    rn = jnp.sqrt(x * x + y * y + z * z)
    inv = 1.0 / (rn + 1e-9)
    ux = x * inv
    uy = y * inv
    uz = z * inv
    shx = _SQ3 * ux
    shy = _SQ3 * uy
    shz = _SQ3 * uz
    a = _SQ3 * ux * uy
    b = _SQ3 * uy * uz
    c = 1.5 * uz * uz - 0.5
    d = _SQ3 * ux * uz
    e = 0.5 * _SQ3 * (ux * ux - uy * uy)
    m00 = e - 0.5 * c
    m11 = -e - 0.5 * c
    zpad = jnp.zeros((7, x.shape[1]), jnp.float32)
    out_ref[...] = jnp.concatenate(
        [shx, shy, shz, m00, a, d, m11, b, c, zpad], axis=0)


def _sh_call(vecT):
    epad = vecT.shape[1]
    ebT = 16384
    grid = epad // ebT
    return pl.pallas_call(
        _sh_body,
        grid=(grid,),
        in_specs=[pl.BlockSpec((3, ebT), lambda i: (0, i))],
        out_specs=pl.BlockSpec((16, ebT), lambda i: (0, i)),
        out_shape=jax.ShapeDtypeStruct((16, epad), jnp.float32),
    )(vecT)


def _sc_body(chained, h_hbm, p_hbm, s_hbm, snd_hbm, rcv_hbm, *rest):
    if chained:
        (yain_hbm, ybin_hbm, ya_hbm, yb_hbm,
         acc_sh, idx_s, idx_r, p_v, s_v, x_v, m_v, z_v,
         sem_p, sem_s, sem_g) = rest
    else:
        (ya_hbm, yb_hbm,
         acc_sh, idx_s, idx_r, p_v, s_v, x_v, m_v, z_v,
         sem_p, sem_s, sem_g) = rest
    core = lax.axis_index("c")
    sub = lax.axis_index("s")
    n = ya_hbm.shape[0]
    rows_per_sub_out = n // _NS
    zrows = z_v.shape[0]
    obase = sub * rows_per_sub_out

    if chained:
        @pl.when(core == 0)
        def _():
            pltpu.sync_copy(yain_hbm.at[pl.ds(obase, rows_per_sub_out)],
                            acc_sh.at[pl.ds(obase, rows_per_sub_out)])

        @pl.when(core == 1)
        def _():
            pltpu.sync_copy(ybin_hbm.at[pl.ds(obase, rows_per_sub_out)],
                            acc_sh.at[pl.ds(obase, rows_per_sub_out)])
    else:
        zero16 = jnp.zeros((16,), jnp.float32)

        def zinit(i, carry):
            z_v[i, 0:16] = zero16
            z_v[i, 16:32] = zero16
            return carry

        lax.fori_loop(0, zrows, zinit, 0)

        def zcopy(i, carry):
            pltpu.sync_copy(z_v, acc_sh.at[pl.ds(obase + i * zrows, zrows)])
            return carry

        lax.fori_loop(0, rows_per_sub_out // zrows, zcopy, 0)
    plsc.subcore_barrier()

    total_rows = snd_hbm.shape[0]
    rows_per_sub = total_rows // _NS
    row0 = sub * rows_per_sub

    def compute_core0(e4, carry):
        for j in range(1):
            e = e4 + j
            x0 = x_v[e, 0:16]
            xx = x_v[e, 16:32]
            xy = x_v[e, 32:48]
            xz = x_v[e, 48:64]
            w0 = p_v[e, 0:16]
            w1 = p_v[e, 16:32]
            w2 = p_v[e, 32:48]
            w3 = p_v[e, 48:64]
            w4 = p_v[e, 64:80]
            sv = s_v[e, 0:16]
            shx = sv[0]
            shy = sv[1]
            shz = sv[2]
            m00 = sv[3]
            m01 = sv[4]
            m02 = sv[5]
            dot = xx * shx + xy * shy + xz * shz
            m0 = w0 * x0 + w1 * dot
            t = w2 * x0
            m1x = t * shx + w3 * xx + w4 * (xx * m00 + xy * m01 + xz * m02)
            m_v[e, 0:16] = m0
            m_v[e, 16:32] = m1x
        return carry

    def compute_core1(e4, carry):
        for j in range(1):
            e = e4 + j
            x0 = x_v[e, 0:16]
            xx = x_v[e, 16:32]
            xy = x_v[e, 32:48]
            xz = x_v[e, 48:64]
            w2 = p_v[e, 32:48]
            w3 = p_v[e, 48:64]
            w4 = p_v[e, 64:80]
            sv = s_v[e, 0:16]
            shy = sv[1]
            shz = sv[2]
            m01 = sv[4]
            m02 = sv[5]
            m11 = sv[6]
            m12 = sv[7]
            m22 = sv[8]
            t = w2 * x0
            m1y = t * shy + w3 * xy + w4 * (xx * m01 + xy * m11 + xz * m12)
            m1z = t * shz + w3 * xz + w4 * (xx * m02 + xy * m12 + xz * m22)
            m_v[e, 0:16] = m1y
            m_v[e, 16:32] = m1z
        return carry

    def chunk(k, carry):
        rbase = row0 + k * _CHUNK_ROWS
        pltpu.sync_copy(snd_hbm.at[pl.ds(rbase, _CHUNK_ROWS)], idx_s)
        pltpu.sync_copy(rcv_hbm.at[pl.ds(rbase, _CHUNK_ROWS)], idx_r)
        for h in range(_CHUNK_ROWS):
            cp = pltpu.async_copy(
                p_hbm.at[pl.ds((rbase + h) * _IDXW, _CE)], p_v, sem_p)
            cs = pltpu.async_copy(
                s_hbm.at[pl.ds((rbase + h) * _IDXW, _CE)], s_v, sem_s)
            gd = pltpu.async_copy(h_hbm.at[idx_s.at[h]], x_v, sem_g)
            cp.wait()
            cs.wait()
            gd.wait()

            @pl.when(core == 0)
            def _():
                lax.fori_loop(0, _CE, compute_core0, 0)

            @pl.when(core == 1)
            def _():
                lax.fori_loop(0, _CE, compute_core1, 0)

            pltpu.sync_copy(m_v, acc_sh.at[idx_r.at[h]], add=True)
        return carry

    lax.fori_loop(0, rows_per_sub // _CHUNK_ROWS, chunk, 0)
    plsc.subcore_barrier()

    @pl.when(core == 0)
    def _():
        pltpu.sync_copy(acc_sh.at[pl.ds(obase, rows_per_sub_out)],
                        ya_hbm.at[pl.ds(obase, rows_per_sub_out)])

    @pl.when(core == 1)
    def _():
        pltpu.sync_copy(acc_sh.at[pl.ds(obase, rows_per_sub_out)],
                        yb_hbm.at[pl.ds(obase, rows_per_sub_out)])


def _sc_call(h_tab, p_edge, s_edge, snd2, rcv2, ya_in=None, yb_in=None):
    mesh = plsc.VectorSubcoreMesh(
        core_axis_name="c", subcore_axis_name="s",
        num_cores=_NC, num_subcores=_NS)
    chained = ya_in is not None
    fn = functools.partial(
        pl.kernel,
        out_type=[
            jax.ShapeDtypeStruct((_NPAD, 32), jnp.float32),
            jax.ShapeDtypeStruct((_NPAD, 32), jnp.float32),
        ],
        mesh=mesh,
        scratch_types=[
            pltpu.VMEM_SHARED((_NPAD, 32), jnp.float32),
            pltpu.VMEM((_CHUNK_ROWS, _IDXW), jnp.int32),
            pltpu.VMEM((_CHUNK_ROWS, _IDXW), jnp.int32),
            pltpu.VMEM((_CE, 80), jnp.float32),
            pltpu.VMEM((_CE, 16), jnp.float32),
            pltpu.VMEM((_CE, 64), jnp.float32),
            pltpu.VMEM((_CE, 32), jnp.float32),
            pltpu.VMEM((8, 32), jnp.float32),
            pltpu.SemaphoreType.DMA,
            pltpu.SemaphoreType.DMA,
            pltpu.SemaphoreType.DMA,
        ],
        compiler_params=pltpu.CompilerParams(use_tc_tiling_on_sc=False),
    )(functools.partial(_sc_body, chained))
    if chained:
        return fn(h_tab, p_edge, s_edge, snd2, rcv2, ya_in, yb_in)
    return fn(h_tab, p_edge, s_edge, snd2, rcv2)


def _node_body(ya_ref, yb_ref, nf0_ref, nf1_ref,
               wd0_ref, wd1_ref, wsk0_ref, wsk1_ref, wsc_ref,
               wp0_ref, wp1_ref, wro_ref,
               z0_ref, z1_ref, ro_ref):
    ya = ya_ref[...]
    yb = yb_ref[...]
    wd0 = wd0_ref[...]
    wd1 = wd1_ref[...]
    s = jnp.dot(ya[:, 0:16], wd0, preferred_element_type=jnp.float32)
    v1x = jnp.dot(ya[:, 16:32], wd1, preferred_element_type=jnp.float32)
    v1y = jnp.dot(yb[:, 0:16], wd1, preferred_element_type=jnp.float32)
    v1z = jnp.dot(yb[:, 16:32], wd1, preferred_element_type=jnp.float32)
    n2 = v1x * v1x + v1y * v1y + v1z * v1z
    wz = wsc_ref[0]
    s2 = s * s
    z0 = (wz[0:1, :] * s + wz[1:2, :] * s2 + wz[2:3, :] * (s2 * s)
          + wz[3:4, :] * n2 + wz[4:5, :] * (s * n2))
    t1 = wz[5:6, :] + wz[6:7, :] * s + wz[7:8, :] * s2 + wz[8:9, :] * n2
    z1x = t1 * v1x
    z1y = t1 * v1y
    z1z = t1 * v1z
    wp0 = wp0_ref[...]
    wp1 = wp1_ref[...]
    z0 = jnp.dot(z0, wp0, preferred_element_type=jnp.float32)
    z1x = jnp.dot(z1x, wp1, preferred_element_type=jnp.float32)
    z1y = jnp.dot(z1y, wp1, preferred_element_type=jnp.float32)
    z1z = jnp.dot(z1z, wp1, preferred_element_type=jnp.float32)
    nf1 = nf1_ref[...]
    wsk0 = wsk0_ref[0]
    wsk1 = wsk1_ref[0]
    z0 = z0 + jnp.dot(nf0_ref[...], wsk0, preferred_element_type=jnp.float32)
    z1x = z1x + jnp.dot(nf1[:, 0:16], wsk1, preferred_element_type=jnp.float32)
    z1y = z1y + jnp.dot(nf1[:, 16:32], wsk1,
                        preferred_element_type=jnp.float32)
    z1z = z1z + jnp.dot(nf1[:, 32:48], wsk1,
                        preferred_element_type=jnp.float32)
    z0_ref[...] = z0
    z1_ref[...] = jnp.concatenate([z1x, z1y, z1z], axis=1)
    ro_ref[...] = jnp.sum(z0 * wro_ref[...][:, 0][None, :], axis=1,
                          keepdims=True)


def _node_call(ya, yb, nf0, nf1p,
               wd0e, wd1e, wsk0, wsk1, wscp, wp0, wp1, wro):
    n = nf0.shape[0]
    nb = min(1000, n // S)
    bps = (n // S) // nb
    grid = n // nb

    return pl.pallas_call(
        _node_body,
        grid=(grid,),
        in_specs=[
            pl.BlockSpec((nb, 32), lambda i: (i, 0)),
            pl.BlockSpec((nb, 32), lambda i: (i, 0)),
            pl.BlockSpec((nb, F), lambda i: (i, 0)),
            pl.BlockSpec((nb, 3 * F), lambda i: (i, 0)),
            pl.BlockSpec((F, F), lambda i: (0, 0)),
            pl.BlockSpec((F, F), lambda i: (0, 0)),
            pl.BlockSpec((1, F, F), lambda i: (i // bps, 0, 0)),
            pl.BlockSpec((1, F, F), lambda i: (i // bps, 0, 0)),
            pl.BlockSpec((1, F, F), lambda i: (i // bps, 0, 0)),
            pl.BlockSpec((F, F), lambda i: (0, 0)),
            pl.BlockSpec((F, F), lambda i: (0, 0)),
            pl.BlockSpec((F, 1), lambda i: (0, 0)),
        ],
        out_specs=[
            pl.BlockSpec((nb, F), lambda i: (i, 0)),
            pl.BlockSpec((nb, 3 * F), lambda i: (i, 0)),
            pl.BlockSpec((nb, 1), lambda i: (i, 0)),
        ],
        out_shape=[
            jax.ShapeDtypeStruct((n, F), jnp.float32),
            jax.ShapeDtypeStruct((n, 3 * F), jnp.float32),
            jax.ShapeDtypeStruct((n, 1), jnp.float32),
        ],
    )(ya, yb, nf0, nf1p,
      wd0e, wd1e, wsk0, wsk1, wscp, wp0, wp1, wro)


def kernel(vectors, node_feats_l0, node_feats_l1, num_species_counts,
           radial_embeddings, senders, receivers, num_nodes,
           W_up_l0, W_up_l1, mlp_w0, mlp_w1, mlp_w2, mlp_w3,
           W_down_l0, W_down_l1, W_skip_l0, W_skip_l1, W_sc,
           W_post_l0, W_post_l1, W_ro):
    n = node_feats_l0.shape[0]
    e = vectors.shape[0]
    epb = _IDXW * _NS * _CHUNK_ROWS
    e_pad = ((e + epb - 1) // epb) * epb

    nf0 = node_feats_l0[:, :, 0]
    nf1p = jnp.transpose(node_feats_l1, (0, 2, 1)).reshape(n, 3 * F)

    h_tab = _up_call(nf0, nf1p, W_up_l0, W_up_l1)

    vecT = jnp.pad(vectors.T, ((0, 0), (0, e_pad - e)))

    zpad_i = jnp.zeros((e_pad - e,), jnp.int32)
    snd2 = jnp.concatenate([senders.astype(jnp.int32), zpad_i]).reshape(
        e_pad // _IDXW, _IDXW)
    rcv2 = jnp.concatenate([receivers.astype(jnp.int32), zpad_i]).reshape(
        e_pad // _IDXW, _IDXW)

    nch = 5
    gr = e_pad // epb
    sizes = [(gr // nch + (1 if i < gr % nch else 0)) * epb
             for i in range(nch)]
    ya = yb = None
    off = 0
    for sz in sizes:
        p_k = _edge_call(radial_embeddings[off:off + sz], sz,
                         mlp_w0, mlp_w1, mlp_w2, mlp_w3)
        s_k = _sh_call(vecT[:, off:off + sz]).T
        r0 = off // _IDXW
        r1 = (off + sz) // _IDXW
        ya, yb = _sc_call(h_tab, p_k, s_k, snd2[r0:r1], rcv2[r0:r1], ya, yb)
        off += sz

    wscp = jnp.concatenate(
        [W_sc, jnp.zeros((S, 7, F), jnp.float32)], axis=1)
    z0f, z1p, ro = _node_call(
        ya, yb, nf0, nf1p, W_down_l0 * EPS, W_down_l1 * EPS,
        W_skip_l0, W_skip_l1, wscp, W_post_l0, W_post_l1, W_ro)

    z0 = z0f[:, :, None]
    z1 = jnp.transpose(z1p.reshape(n, 3, F), (0, 2, 1))
    return z0, z1, ro
